# Initial kernel scaffold; baseline (speedup 1.0000x reference)
import jax
import jax.numpy as jnp
from jax.experimental import pallas as pl


def kernel(x, edge_index, W1, a1_src, a1_dst, b1, W2, a2_src, a2_dst, b2):
    raise NotImplementedError("write your pallas kernel here")


# speedup vs baseline: 61.9961x; 61.9961x over previous
import functools

import jax
import jax.numpy as jnp
import numpy as np
from jax import lax
from jax.experimental import pallas as pl
from jax.experimental.pallas import tpu as pltpu
from jax.experimental.pallas import tpu_sc as plsc

N = 10000
E = 320000
F_IN = 128

NPAD = 10240
NC, NS = 2, 16
NW = NC * NS
EB = 128
EPW = 10112
NB = EPW // EB
ROWS_PER_SUB = NPAD // NS
RCHUNK = 128
NRC = ROWS_PER_SUB // RCHUNK
W1T = 80
W2T = 32

_NBLK = 512
_GRID = NPAD // _NBLK

_SC_PARAMS = pltpu.CompilerParams(use_tc_tiling_on_sc=False,
                                  needs_layout_passes=False)


def _tc1_body(x_ref, w1_ref, ams_ref, amd_ref, tab_ref, adt_ref):
    h = jnp.dot(x_ref[...], w1_ref[...], preferred_element_type=jnp.float32)
    asb = jnp.dot(h, ams_ref[...], preferred_element_type=jnp.float32)
    adb = jnp.dot(h, amd_ref[...], preferred_element_type=jnp.float32)
    tab_ref[...] = jnp.concatenate([h, asb], axis=1)
    adt_ref[...] = adb


def _tc1(xp, W1, AMs, AMd):
    return pl.pallas_call(
        _tc1_body,
        grid=(_GRID,),
        in_specs=[
            pl.BlockSpec((_NBLK, F_IN), lambda i: (i, 0)),
            pl.BlockSpec((F_IN, 64), lambda i: (0, 0)),
            pl.BlockSpec((64, 16), lambda i: (0, 0)),
            pl.BlockSpec((64, 16), lambda i: (0, 0)),
        ],
        out_specs=[
            pl.BlockSpec((_NBLK, W1T), lambda i: (i, 0)),
            pl.BlockSpec((_NBLK, 16), lambda i: (i, 0)),
        ],
        out_shape=[
            jax.ShapeDtypeStruct((NPAD, W1T), jnp.float32),
            jax.ShapeDtypeStruct((NPAD, 16), jnp.float32),
        ],
    )(xp, W1, AMs, AMd)


def _tc2_body(acc_ref, b1_ref, w2_ref, ms_ref, md_ref, r_ref,
              tab_ref, as2_ref, ad2_ref):
    msg = acc_ref[0, :, :64] + acc_ref[1, :, :64]
    den16 = acc_ref[0, :, 64:80] + acc_ref[1, :, 64:80]
    den = jnp.dot(den16, r_ref[...], preferred_element_type=jnp.float32)
    hl = msg / (den + 1e-16) + b1_ref[...]
    hl = jnp.where(hl > 0, hl, jnp.exp(hl) - 1.0)
    h2 = jnp.dot(hl, w2_ref[...], preferred_element_type=jnp.float32)
    asb = jnp.dot(h2, ms_ref[...], preferred_element_type=jnp.float32)
    adb = jnp.dot(h2, md_ref[...], preferred_element_type=jnp.float32)
    tab_ref[...] = h2
    as2_ref[...] = asb
    ad2_ref[...] = adb


def _tc2(acc1, b1, W2, Ms2, Md2, R):
    return pl.pallas_call(
        _tc2_body,
        grid=(_GRID,),
        in_specs=[
            pl.BlockSpec((2, _NBLK, W1T), lambda i: (0, i, 0)),
            pl.BlockSpec((1, 64), lambda i: (0, 0)),
            pl.BlockSpec((64, 16), lambda i: (0, 0)),
            pl.BlockSpec((16, 16), lambda i: (0, 0)),
            pl.BlockSpec((16, 16), lambda i: (0, 0)),
            pl.BlockSpec((16, 64), lambda i: (0, 0)),
        ],
        out_specs=[
            pl.BlockSpec((_NBLK, 16), lambda i: (i, 0)),
            pl.BlockSpec((_NBLK, 16), lambda i: (i, 0)),
            pl.BlockSpec((_NBLK, 16), lambda i: (i, 0)),
        ],
        out_shape=[
            jax.ShapeDtypeStruct((NPAD, 16), jnp.float32),
            jax.ShapeDtypeStruct((NPAD, 16), jnp.float32),
            jax.ShapeDtypeStruct((NPAD, 16), jnp.float32),
        ],
    )(acc1, b1, W2, Ms2, Md2, R)


def _tc3_body(acc_ref, b2_ref, out_ref):
    msg = acc_ref[0, :, :16] + acc_ref[1, :, :16]
    den = acc_ref[0, :, 16:32] + acc_ref[1, :, 16:32]
    o = msg / (den + 1e-16) + b2_ref[...]
    m = jnp.max(o, axis=1, keepdims=True)
    z = o - m
    out_ref[...] = z - jnp.log(jnp.sum(jnp.exp(z), axis=1, keepdims=True))


def _tc3(acc2, b2):
    return pl.pallas_call(
        _tc3_body,
        grid=(_GRID,),
        in_specs=[
            pl.BlockSpec((2, _NBLK, W2T), lambda i: (0, i, 0)),
            pl.BlockSpec((1, 16), lambda i: (0, 0)),
        ],
        out_specs=pl.BlockSpec((_NBLK, 16), lambda i: (i, 0)),
        out_shape=jax.ShapeDtypeStruct((NPAD, 16), jnp.float32),
    )(acc2, b2)


_GDN = lax.GatherDimensionNumbers(
    offset_dims=(), collapsed_slice_dims=(0,), start_index_map=(0,))


def _bcast_pair(p, k):
    io = lax.iota(jnp.int32, 16)
    idx = 2 * k + jnp.where(io >= 8, 1, 0)
    return lax.gather(p, idx.reshape(16, 1), _GDN, (1,),
                      mode=lax.GatherScatterMode.PROMISE_IN_BOUNDS)


def _leaky_exp(e):
    return jnp.exp(jnp.maximum(e, e * 0.2))


def _zero_accum(orows, accum, s, w):
    zv = jnp.zeros((16,), jnp.float32)

    def zrow(i, _):
        for kk in range(w // 16):
            orows[i, pl.ds(kk * 16, 16)] = zv
        return 0

    lax.fori_loop(0, EB, zrow, 0)
    base = s * ROWS_PER_SUB
    for jj in range(NRC):
        pltpu.sync_copy(orows, accum.at[pl.ds(base + jj * RCHUNK, RCHUNK)])
    return base


def _dump_accum(accum, out_ref, c, base):
    for jj in range(NRC):
        r0 = base + jj * RCHUNK
        pltpu.sync_copy(accum.at[pl.ds(r0, RCHUNK)],
                        out_ref.at[c, pl.ds(r0, RCHUNK)])


def _edge_pass1(tab, adt, esrc, edst):
    mesh = plsc.VectorSubcoreMesh(core_axis_name="c", subcore_axis_name="s")

    @functools.partial(
        pl.kernel,
        out_type=jax.ShapeDtypeStruct((NC, NPAD, W1T), jnp.float32),
        mesh=mesh,
        compiler_params=_SC_PARAMS,
        scratch_types=[
            pltpu.VMEM((NB, EB), jnp.int32),
            pltpu.VMEM((NB, EB), jnp.int32),
            pltpu.VMEM((EB, W1T), jnp.float32),
            pltpu.VMEM((EB, 16), jnp.float32),
            pltpu.VMEM((EB, W1T), jnp.float32),
            pltpu.VMEM_SHARED((NPAD, W1T), jnp.float32),
        ],
    )
    def k(tab_ref, adt_ref, esrc_ref, edst_ref, out_ref,
          idxs, idxd, srows, drows, orows, accum):
        c = lax.axis_index("c")
        s = lax.axis_index("s")
        wid = c * NS + s
        pltpu.sync_copy(esrc_ref.at[wid], idxs)
        pltpu.sync_copy(edst_ref.at[wid], idxd)
        base = _zero_accum(orows, accum, s, W1T)
        plsc.subcore_barrier()

        def block(b, _):
            pltpu.sync_copy(tab_ref.at[idxs.at[b]], srows)
            pltpu.sync_copy(adt_ref.at[idxd.at[b]], drows)

            def edge(j, _):
                sa = srows[j, pl.ds(64, 16)]
                da = drows[j, pl.ds(0, 16)]
                p = _leaky_exp(sa + da)
                for kk in range(4):
                    hk = srows[j, pl.ds(kk * 16, 16)]
                    orows[j, pl.ds(kk * 16, 16)] = hk * _bcast_pair(p, kk)
                orows[j, pl.ds(64, 16)] = p
                return 0

            lax.fori_loop(0, EB, edge, 0)
            pltpu.sync_copy(orows, accum.at[idxd.at[b]], add=True)
            return 0

        lax.fori_loop(0, NB, block, 0)
        plsc.subcore_barrier()
        _dump_accum(accum, out_ref, c, base)

    return k(tab, adt, esrc, edst)


def _edge_pass2(tab, as2, ad2, esrc, edst):
    mesh = plsc.VectorSubcoreMesh(core_axis_name="c", subcore_axis_name="s")

    @functools.partial(
        pl.kernel,
        out_type=jax.ShapeDtypeStruct((NC, NPAD, W2T), jnp.float32),
        mesh=mesh,
        compiler_params=_SC_PARAMS,
        scratch_types=[
            pltpu.VMEM((NB, EB), jnp.int32),
            pltpu.VMEM((NB, EB), jnp.int32),
            pltpu.VMEM((EB, 16), jnp.float32),
            pltpu.VMEM((EB, W2T), jnp.float32),
            pltpu.VMEM((NPAD,), jnp.float32),
            pltpu.VMEM((NPAD,), jnp.float32),
            pltpu.VMEM((EB,), jnp.float32),
            pltpu.VMEM_SHARED((NPAD, W2T), jnp.float32),
        ],
    )
    def k(tab_ref, as2_ref, ad2_ref, esrc_ref, edst_ref, out_ref,
          idxs, idxd, srows, orows, asv, adv, pbuf, accum):
        c = lax.axis_index("c")
        s = lax.axis_index("s")
        wid = c * NS + s
        pltpu.sync_copy(esrc_ref.at[wid], idxs)
        pltpu.sync_copy(edst_ref.at[wid], idxd)
        pltpu.sync_copy(as2_ref, asv)
        pltpu.sync_copy(ad2_ref, adv)
        base = _zero_accum(orows, accum, s, W2T)
        plsc.subcore_barrier()

        def block(b, _):
            pltpu.sync_copy(tab_ref.at[idxs.at[b]], srows)

            def grp(g, _):
                srcv = idxs[b, pl.ds(g * 16, 16)]
                dstv = idxd[b, pl.ds(g * 16, 16)]
                e = (plsc.load_gather(asv, [srcv]) +
                     plsc.load_gather(adv, [dstv]))
                pbuf[pl.ds(g * 16, 16)] = _leaky_exp(e)
                return 0

            lax.fori_loop(0, EB // 16, grp, 0)

            def edge(j, _):
                pv = plsc.load_gather(pbuf, [jnp.broadcast_to(j, (16,))])
                orows[j, pl.ds(0, 16)] = srows[j, pl.ds(0, 16)] * pv
                orows[j, pl.ds(16, 16)] = pv
                return 0

            lax.fori_loop(0, EB, edge, 0)
            pltpu.sync_copy(orows, accum.at[idxd.at[b]], add=True)
            return 0

        lax.fori_loop(0, NB, block, 0)
        plsc.subcore_barrier()
        _dump_accum(accum, out_ref, c, base)

    return k(tab, as2, ad2, esrc, edst)


def kernel(x, edge_index, W1, a1_src, a1_dst, b1, W2, a2_src, a2_dst, b2):
    eye8 = jnp.eye(8, dtype=jnp.float32)
    AMs = jnp.pad((a1_src[:, :, None] * eye8[:, None, :]).reshape(64, 8),
                  ((0, 0), (0, 8)))
    AMd = jnp.pad((a1_dst[:, :, None] * eye8[:, None, :]).reshape(64, 8),
                  ((0, 0), (0, 8)))
    R = jnp.pad(jnp.repeat(eye8, 8, axis=1), ((0, 8), (0, 0)))
    ones16 = jnp.ones((1, 16), dtype=jnp.float32)
    Ms2 = a2_src.reshape(16, 1) @ ones16
    Md2 = a2_dst.reshape(16, 1) @ ones16

    xp = jnp.pad(x, ((0, NPAD - N), (0, 0)))
    tab1, adt1 = _tc1(xp, W1, AMs, AMd)

    ei = jnp.pad(edge_index, ((0, 0), (0, NW * EPW - E)), constant_values=N)
    esrc = ei[0].reshape(NW, NB, EB)
    edst = ei[1].reshape(NW, NB, EB)

    acc1 = _edge_pass1(tab1, adt1, esrc, edst)
    tab2, as2b, ad2b = _tc2(acc1, b1.reshape(1, 64), W2, Ms2, Md2, R)
    acc2 = _edge_pass2(tab2, as2b[:, 0], ad2b[:, 0], esrc, edst)
    out = _tc3(acc2, b2.reshape(1, 16))
    return out[:N]


# speedup vs baseline: 79.5500x; 1.2831x over previous
import functools

import jax
import jax.numpy as jnp
import numpy as np
from jax import lax
from jax.experimental import pallas as pl
from jax.experimental.pallas import tpu as pltpu
from jax.experimental.pallas import tpu_sc as plsc

N = 10000
E = 320000
F_IN = 128

NPAD = 10240
NC, NS = 2, 16
NW = NC * NS
EB = 128
EPW = 10112
NB = EPW // EB
ROWS_PER_SUB = NPAD // NS
RCHUNK = 128
NRC = ROWS_PER_SUB // RCHUNK
W1T = 80
W2T = 32

_NBLK = 512
_GRID = NPAD // _NBLK

_SC_PARAMS = pltpu.CompilerParams(use_tc_tiling_on_sc=False,
                                  needs_layout_passes=False)


def _tc1_body(x_ref, w1_ref, ams_ref, amd_ref, tab_ref, adt_ref):
    h = jnp.dot(x_ref[...], w1_ref[...], preferred_element_type=jnp.float32)
    asb = jnp.dot(h, ams_ref[...], preferred_element_type=jnp.float32)
    adb = jnp.dot(h, amd_ref[...], preferred_element_type=jnp.float32)
    tab_ref[...] = jnp.concatenate([h, asb], axis=1)
    adt_ref[...] = adb


def _tc1(xp, W1, AMs, AMd):
    return pl.pallas_call(
        _tc1_body,
        grid=(_GRID,),
        in_specs=[
            pl.BlockSpec((_NBLK, F_IN), lambda i: (i, 0)),
            pl.BlockSpec((F_IN, 64), lambda i: (0, 0)),
            pl.BlockSpec((64, 16), lambda i: (0, 0)),
            pl.BlockSpec((64, 16), lambda i: (0, 0)),
        ],
        out_specs=[
            pl.BlockSpec((_NBLK, W1T), lambda i: (i, 0)),
            pl.BlockSpec((_NBLK, 16), lambda i: (i, 0)),
        ],
        out_shape=[
            jax.ShapeDtypeStruct((NPAD, W1T), jnp.float32),
            jax.ShapeDtypeStruct((NPAD, 16), jnp.float32),
        ],
    )(xp, W1, AMs, AMd)


def _tc2_body(acc_ref, b1_ref, w2_ref, ms_ref, md_ref, r_ref,
              tab_ref, as2_ref, ad2_ref):
    msg = acc_ref[0, :, :64] + acc_ref[1, :, :64]
    den16 = acc_ref[0, :, 64:80] + acc_ref[1, :, 64:80]
    den = jnp.dot(den16, r_ref[...], preferred_element_type=jnp.float32)
    hl = msg / (den + 1e-16) + b1_ref[...]
    hl = jnp.where(hl > 0, hl, jnp.exp(hl) - 1.0)
    h2 = jnp.dot(hl, w2_ref[...], preferred_element_type=jnp.float32)
    asb = jnp.dot(h2, ms_ref[...], preferred_element_type=jnp.float32)
    adb = jnp.dot(h2, md_ref[...], preferred_element_type=jnp.float32)
    tab_ref[...] = h2
    as2_ref[...] = asb
    ad2_ref[...] = adb


def _tc2(acc1, b1, W2, Ms2, Md2, R):
    return pl.pallas_call(
        _tc2_body,
        grid=(_GRID,),
        in_specs=[
            pl.BlockSpec((2, _NBLK, W1T), lambda i: (0, i, 0)),
            pl.BlockSpec((1, 64), lambda i: (0, 0)),
            pl.BlockSpec((64, 16), lambda i: (0, 0)),
            pl.BlockSpec((16, 16), lambda i: (0, 0)),
            pl.BlockSpec((16, 16), lambda i: (0, 0)),
            pl.BlockSpec((16, 64), lambda i: (0, 0)),
        ],
        out_specs=[
            pl.BlockSpec((_NBLK, 16), lambda i: (i, 0)),
            pl.BlockSpec((_NBLK, 16), lambda i: (i, 0)),
            pl.BlockSpec((_NBLK, 16), lambda i: (i, 0)),
        ],
        out_shape=[
            jax.ShapeDtypeStruct((NPAD, 16), jnp.float32),
            jax.ShapeDtypeStruct((NPAD, 16), jnp.float32),
            jax.ShapeDtypeStruct((NPAD, 16), jnp.float32),
        ],
    )(acc1, b1, W2, Ms2, Md2, R)


def _tc3_body(acc_ref, b2_ref, out_ref):
    msg = acc_ref[0, :, :16] + acc_ref[1, :, :16]
    den = acc_ref[0, :, 16:32] + acc_ref[1, :, 16:32]
    o = msg / (den + 1e-16) + b2_ref[...]
    m = jnp.max(o, axis=1, keepdims=True)
    z = o - m
    out_ref[...] = z - jnp.log(jnp.sum(jnp.exp(z), axis=1, keepdims=True))


def _tc3(acc2, b2):
    return pl.pallas_call(
        _tc3_body,
        grid=(_GRID,),
        in_specs=[
            pl.BlockSpec((2, _NBLK, W2T), lambda i: (0, i, 0)),
            pl.BlockSpec((1, 16), lambda i: (0, 0)),
        ],
        out_specs=pl.BlockSpec((_NBLK, 16), lambda i: (i, 0)),
        out_shape=jax.ShapeDtypeStruct((NPAD, 16), jnp.float32),
    )(acc2, b2)


_GDN = lax.GatherDimensionNumbers(
    offset_dims=(), collapsed_slice_dims=(0,), start_index_map=(0,))


def _bcast_pair(p, k):
    io = lax.iota(jnp.int32, 16)
    idx = 2 * k + jnp.where(io >= 8, 1, 0)
    return lax.gather(p, idx.reshape(16, 1), _GDN, (1,),
                      mode=lax.GatherScatterMode.PROMISE_IN_BOUNDS)


def _leaky_exp(e):
    return jnp.exp(jnp.maximum(e, e * 0.2))


def _zero_accum(orows, accum, s, w):
    zv = jnp.zeros((16,), jnp.float32)

    def zrow(i, _):
        for kk in range(w // 16):
            orows[i, pl.ds(kk * 16, 16)] = zv
        return 0

    lax.fori_loop(0, EB, zrow, 0)
    base = s * ROWS_PER_SUB
    for jj in range(NRC):
        pltpu.sync_copy(orows, accum.at[pl.ds(base + jj * RCHUNK, RCHUNK)])
    return base


def _dump_accum(accum, out_ref, c, base):
    for jj in range(NRC):
        r0 = base + jj * RCHUNK
        pltpu.sync_copy(accum.at[pl.ds(r0, RCHUNK)],
                        out_ref.at[c, pl.ds(r0, RCHUNK)])


def _edge_pass1(tab, adt, esrc, edst):
    mesh = plsc.VectorSubcoreMesh(core_axis_name="c", subcore_axis_name="s")

    @functools.partial(
        pl.kernel,
        out_type=jax.ShapeDtypeStruct((NC, NPAD, W1T), jnp.float32),
        mesh=mesh,
        compiler_params=_SC_PARAMS,
        scratch_types=[
            pltpu.VMEM((NB, EB), jnp.int32),
            pltpu.VMEM((NB, EB), jnp.int32),
            pltpu.VMEM((EB, W1T), jnp.float32),
            pltpu.VMEM((EB, W1T), jnp.float32),
            pltpu.VMEM((EB, 16), jnp.float32),
            pltpu.VMEM((EB, 16), jnp.float32),
            pltpu.VMEM((EB, W1T), jnp.float32),
            pltpu.SemaphoreType.DMA,
            pltpu.SemaphoreType.DMA,
            pltpu.VMEM_SHARED((NPAD, W1T), jnp.float32),
        ],
    )
    def k(tab_ref, adt_ref, esrc_ref, edst_ref, out_ref,
          idxs, idxd, srows0, srows1, drows0, drows1, orows,
          gsem0, gsem1, accum):
        c = lax.axis_index("c")
        s = lax.axis_index("s")
        wid = c * NS + s
        pltpu.sync_copy(esrc_ref.at[wid], idxs)
        pltpu.sync_copy(edst_ref.at[wid], idxd)
        base = _zero_accum(orows, accum, s, W1T)
        plsc.subcore_barrier()

        bufs = ((srows0, drows0, gsem0), (srows1, drows1, gsem1))

        def g_start(b, i):
            sb, db, sem = bufs[i]
            pltpu.async_copy(tab_ref.at[idxs.at[b]], sb, sem)
            pltpu.async_copy(adt_ref.at[idxd.at[b]], db, sem)

        def g_wait(b, i):
            sb, db, sem = bufs[i]
            pltpu.make_async_copy(tab_ref.at[idxs.at[b]], sb, sem).wait()
            pltpu.make_async_copy(adt_ref.at[idxd.at[b]], db, sem).wait()

        def compute_block(b, i):
            sb, db, _ = bufs[i]

            def edge(j, _):
                sa = sb[j, pl.ds(64, 16)]
                da = db[j, pl.ds(0, 16)]
                p = _leaky_exp(sa + da)
                for kk in range(4):
                    hk = sb[j, pl.ds(kk * 16, 16)]
                    orows[j, pl.ds(kk * 16, 16)] = hk * _bcast_pair(p, kk)
                orows[j, pl.ds(64, 16)] = p
                return 0

            lax.fori_loop(0, EB, edge, 0, unroll=4)
            pltpu.sync_copy(orows, accum.at[idxd.at[b]], add=True)

        g_start(0, 0)

        def pair(g2, _):
            for i in range(2):
                b = 2 * g2 + i

                @pl.when(b < NB)
                def _():
                    g_wait(b, i)

                    @pl.when(b + 1 < NB)
                    def _():
                        g_start(b + 1, 1 - i)

                    compute_block(b, i)
            return 0

        lax.fori_loop(0, (NB + 1) // 2, pair, 0)
        plsc.subcore_barrier()
        _dump_accum(accum, out_ref, c, base)

    return k(tab, adt, esrc, edst)


def _edge_pass2(tab, as2, ad2, esrc, edst):
    mesh = plsc.VectorSubcoreMesh(core_axis_name="c", subcore_axis_name="s")

    @functools.partial(
        pl.kernel,
        out_type=jax.ShapeDtypeStruct((NC, NPAD, W2T), jnp.float32),
        mesh=mesh,
        compiler_params=_SC_PARAMS,
        scratch_types=[
            pltpu.VMEM((NB, EB), jnp.int32),
            pltpu.VMEM((NB, EB), jnp.int32),
            pltpu.VMEM((EB, 16), jnp.float32),
            pltpu.VMEM((EB, 16), jnp.float32),
            pltpu.VMEM((EB, W2T), jnp.float32),
            pltpu.VMEM((NPAD,), jnp.float32),
            pltpu.VMEM((NPAD,), jnp.float32),
            pltpu.VMEM((EB,), jnp.float32),
            pltpu.SemaphoreType.DMA,
            pltpu.SemaphoreType.DMA,
            pltpu.VMEM_SHARED((NPAD, W2T), jnp.float32),
        ],
    )
    def k(tab_ref, as2_ref, ad2_ref, esrc_ref, edst_ref, out_ref,
          idxs, idxd, srows0, srows1, orows, asv, adv, pbuf,
          gsem0, gsem1, accum):
        c = lax.axis_index("c")
        s = lax.axis_index("s")
        wid = c * NS + s
        pltpu.sync_copy(esrc_ref.at[wid], idxs)
        pltpu.sync_copy(edst_ref.at[wid], idxd)
        pltpu.sync_copy(as2_ref, asv)
        pltpu.sync_copy(ad2_ref, adv)
        base = _zero_accum(orows, accum, s, W2T)
        plsc.subcore_barrier()

        bufs = ((srows0, gsem0), (srows1, gsem1))

        def g_start(b, i):
            sb, sem = bufs[i]
            pltpu.async_copy(tab_ref.at[idxs.at[b]], sb, sem)

        def g_wait(b, i):
            sb, sem = bufs[i]
            pltpu.make_async_copy(tab_ref.at[idxs.at[b]], sb, sem).wait()

        def compute_block(b, i):
            sb, _ = bufs[i]

            def grp(g, _):
                srcv = idxs[b, pl.ds(g * 16, 16)]
                dstv = idxd[b, pl.ds(g * 16, 16)]
                e = (plsc.load_gather(asv, [srcv]) +
                     plsc.load_gather(adv, [dstv]))
                pbuf[pl.ds(g * 16, 16)] = _leaky_exp(e)
                return 0

            lax.fori_loop(0, EB // 16, grp, 0, unroll=2)

            def edge(j, _):
                pv = plsc.load_gather(pbuf, [jnp.broadcast_to(j, (16,))])
                orows[j, pl.ds(0, 16)] = sb[j, pl.ds(0, 16)] * pv
                orows[j, pl.ds(16, 16)] = pv
                return 0

            lax.fori_loop(0, EB, edge, 0, unroll=4)
            pltpu.sync_copy(orows, accum.at[idxd.at[b]], add=True)

        g_start(0, 0)

        def pair(g2, _):
            for i in range(2):
                b = 2 * g2 + i

                @pl.when(b < NB)
                def _():
                    g_wait(b, i)

                    @pl.when(b + 1 < NB)
                    def _():
                        g_start(b + 1, 1 - i)

                    compute_block(b, i)
            return 0

        lax.fori_loop(0, (NB + 1) // 2, pair, 0)
        plsc.subcore_barrier()
        _dump_accum(accum, out_ref, c, base)

    return k(tab, as2, ad2, esrc, edst)


def kernel(x, edge_index, W1, a1_src, a1_dst, b1, W2, a2_src, a2_dst, b2):
    eye8 = jnp.eye(8, dtype=jnp.float32)
    AMs = jnp.pad((a1_src[:, :, None] * eye8[:, None, :]).reshape(64, 8),
                  ((0, 0), (0, 8)))
    AMd = jnp.pad((a1_dst[:, :, None] * eye8[:, None, :]).reshape(64, 8),
                  ((0, 0), (0, 8)))
    R = jnp.pad(jnp.repeat(eye8, 8, axis=1), ((0, 8), (0, 0)))
    ones16 = jnp.ones((1, 16), dtype=jnp.float32)
    Ms2 = a2_src.reshape(16, 1) @ ones16
    Md2 = a2_dst.reshape(16, 1) @ ones16

    xp = jnp.pad(x, ((0, NPAD - N), (0, 0)))
    tab1, adt1 = _tc1(xp, W1, AMs, AMd)

    ei = jnp.pad(edge_index, ((0, 0), (0, NW * EPW - E)), constant_values=N)
    esrc = ei[0].reshape(NW, NB, EB)
    edst = ei[1].reshape(NW, NB, EB)

    acc1 = _edge_pass1(tab1, adt1, esrc, edst)
    tab2, as2b, ad2b = _tc2(acc1, b1.reshape(1, 64), W2, Ms2, Md2, R)
    acc2 = _edge_pass2(tab2, as2b[:, 0], ad2b[:, 0], esrc, edst)
    out = _tc3(acc2, b2.reshape(1, 16))
    return out[:N]


# speedup vs baseline: 83.6036x; 1.0510x over previous
import functools

import jax
import jax.numpy as jnp
import numpy as np
from jax import lax
from jax.experimental import pallas as pl
from jax.experimental.pallas import tpu as pltpu
from jax.experimental.pallas import tpu_sc as plsc

N = 10000
E = 320000
F_IN = 128

NPAD = 10240
NC, NS = 2, 16
NW = NC * NS
EB = 128
EPW = 10112
NB = EPW // EB
ROWS_PER_SUB = NPAD // NS
RCHUNK = 128
NRC = ROWS_PER_SUB // RCHUNK
W1T = 80
W2T = 32

_NBLK = 512
_GRID = NPAD // _NBLK

_SC_PARAMS = pltpu.CompilerParams(use_tc_tiling_on_sc=False,
                                  needs_layout_passes=False)


def _tc1_body(x_ref, w1_ref, ams_ref, amd_ref, tab_ref, adt_ref):
    h = jnp.dot(x_ref[...], w1_ref[...], preferred_element_type=jnp.float32)
    asb = jnp.dot(h, ams_ref[...], preferred_element_type=jnp.float32)
    adb = jnp.dot(h, amd_ref[...], preferred_element_type=jnp.float32)
    tab_ref[...] = jnp.concatenate([h, asb], axis=1)
    adt_ref[...] = adb


def _tc1(xp, W1, AMs, AMd):
    return pl.pallas_call(
        _tc1_body,
        grid=(_GRID,),
        in_specs=[
            pl.BlockSpec((_NBLK, F_IN), lambda i: (i, 0)),
            pl.BlockSpec((F_IN, 64), lambda i: (0, 0)),
            pl.BlockSpec((64, 16), lambda i: (0, 0)),
            pl.BlockSpec((64, 16), lambda i: (0, 0)),
        ],
        out_specs=[
            pl.BlockSpec((_NBLK, W1T), lambda i: (i, 0)),
            pl.BlockSpec((_NBLK, 16), lambda i: (i, 0)),
        ],
        out_shape=[
            jax.ShapeDtypeStruct((NPAD, W1T), jnp.float32),
            jax.ShapeDtypeStruct((NPAD, 16), jnp.float32),
        ],
    )(xp, W1, AMs, AMd)


def _tc2_body(acc_ref, b1_ref, w2_ref, ms_ref, md_ref, r_ref,
              tab_ref, as2_ref, ad2_ref):
    msg = acc_ref[0, :, :64] + acc_ref[1, :, :64]
    den16 = acc_ref[0, :, 64:80] + acc_ref[1, :, 64:80]
    den = jnp.dot(den16, r_ref[...], preferred_element_type=jnp.float32)
    hl = msg / (den + 1e-16) + b1_ref[...]
    hl = jnp.where(hl > 0, hl, jnp.exp(hl) - 1.0)
    h2 = jnp.dot(hl, w2_ref[...], preferred_element_type=jnp.float32)
    asb = jnp.dot(h2, ms_ref[...], preferred_element_type=jnp.float32)
    adb = jnp.dot(h2, md_ref[...], preferred_element_type=jnp.float32)
    tab_ref[...] = h2
    as2_ref[...] = asb
    ad2_ref[...] = adb


def _tc2(acc1, b1, W2, Ms2, Md2, R):
    return pl.pallas_call(
        _tc2_body,
        grid=(_GRID,),
        in_specs=[
            pl.BlockSpec((2, _NBLK, W1T), lambda i: (0, i, 0)),
            pl.BlockSpec((1, 64), lambda i: (0, 0)),
            pl.BlockSpec((64, 16), lambda i: (0, 0)),
            pl.BlockSpec((16, 16), lambda i: (0, 0)),
            pl.BlockSpec((16, 16), lambda i: (0, 0)),
            pl.BlockSpec((16, 64), lambda i: (0, 0)),
        ],
        out_specs=[
            pl.BlockSpec((_NBLK, 16), lambda i: (i, 0)),
            pl.BlockSpec((_NBLK, 16), lambda i: (i, 0)),
            pl.BlockSpec((_NBLK, 16), lambda i: (i, 0)),
        ],
        out_shape=[
            jax.ShapeDtypeStruct((NPAD, 16), jnp.float32),
            jax.ShapeDtypeStruct((NPAD, 16), jnp.float32),
            jax.ShapeDtypeStruct((NPAD, 16), jnp.float32),
        ],
    )(acc1, b1, W2, Ms2, Md2, R)


def _tc3_body(acc_ref, b2_ref, out_ref):
    msg = acc_ref[0, :, :16] + acc_ref[1, :, :16]
    den = acc_ref[0, :, 16:32] + acc_ref[1, :, 16:32]
    o = msg / (den + 1e-16) + b2_ref[...]
    m = jnp.max(o, axis=1, keepdims=True)
    z = o - m
    out_ref[...] = z - jnp.log(jnp.sum(jnp.exp(z), axis=1, keepdims=True))


def _tc3(acc2, b2):
    return pl.pallas_call(
        _tc3_body,
        grid=(_GRID,),
        in_specs=[
            pl.BlockSpec((2, _NBLK, W2T), lambda i: (0, i, 0)),
            pl.BlockSpec((1, 16), lambda i: (0, 0)),
        ],
        out_specs=pl.BlockSpec((_NBLK, 16), lambda i: (i, 0)),
        out_shape=jax.ShapeDtypeStruct((NPAD, 16), jnp.float32),
    )(acc2, b2)


_GDN = lax.GatherDimensionNumbers(
    offset_dims=(), collapsed_slice_dims=(0,), start_index_map=(0,))


def _pair_idx():
    io = lax.iota(jnp.int32, 16)
    hi = jnp.where(io >= 8, 1, 0)
    return [(2 * k + hi).reshape(16, 1) for k in range(4)]


def _bcast_pair(p, idx):
    return lax.gather(p, idx, _GDN, (1,),
                      mode=lax.GatherScatterMode.PROMISE_IN_BOUNDS)


def _leaky_exp(e):
    return jnp.exp(jnp.maximum(e, e * 0.2))


def _zero_accum(orows, accum, s, w):
    zv = jnp.zeros((16,), jnp.float32)

    def zrow(i, _):
        for kk in range(w // 16):
            orows[i, pl.ds(kk * 16, 16)] = zv
        return 0

    lax.fori_loop(0, EB, zrow, 0)
    base = s * ROWS_PER_SUB
    for jj in range(NRC):
        pltpu.sync_copy(orows, accum.at[pl.ds(base + jj * RCHUNK, RCHUNK)])
    return base


def _dump_accum(accum, out_ref, c, base):
    for jj in range(NRC):
        r0 = base + jj * RCHUNK
        pltpu.sync_copy(accum.at[pl.ds(r0, RCHUNK)],
                        out_ref.at[c, pl.ds(r0, RCHUNK)])


def _edge_pass1(tab, adt, esrc, edst):
    mesh = plsc.VectorSubcoreMesh(core_axis_name="c", subcore_axis_name="s")

    @functools.partial(
        pl.kernel,
        out_type=jax.ShapeDtypeStruct((NC, NPAD, W1T), jnp.float32),
        mesh=mesh,
        compiler_params=_SC_PARAMS,
        scratch_types=[
            pltpu.VMEM((NB, EB), jnp.int32),
            pltpu.VMEM((NB, EB), jnp.int32),
            pltpu.VMEM((EB, W1T), jnp.float32),
            pltpu.VMEM((EB, W1T), jnp.float32),
            pltpu.VMEM((EB, 16), jnp.float32),
            pltpu.VMEM((EB, 16), jnp.float32),
            pltpu.VMEM((EB, W1T), jnp.float32),
            pltpu.VMEM((EB, W1T), jnp.float32),
            pltpu.SemaphoreType.DMA,
            pltpu.SemaphoreType.DMA,
            pltpu.SemaphoreType.DMA,
            pltpu.SemaphoreType.DMA,
            pltpu.VMEM_SHARED((NPAD, W1T), jnp.float32),
        ],
    )
    def k(tab_ref, adt_ref, esrc_ref, edst_ref, out_ref,
          idxs, idxd, srows0, srows1, drows0, drows1, orows0, orows1,
          gsem0, gsem1, ssem0, ssem1, accum):
        c = lax.axis_index("c")
        s = lax.axis_index("s")
        wid = c * NS + s
        pltpu.sync_copy(esrc_ref.at[wid], idxs)
        pltpu.sync_copy(edst_ref.at[wid], idxd)
        base = _zero_accum(orows0, accum, s, W1T)
        plsc.subcore_barrier()

        bufs = ((srows0, drows0, orows0, gsem0, ssem0),
                (srows1, drows1, orows1, gsem1, ssem1))
        pidx = _pair_idx()

        def g_start(b, i):
            sb, db, _, sem, _ = bufs[i]
            pltpu.async_copy(tab_ref.at[idxs.at[b]], sb, sem)
            pltpu.async_copy(adt_ref.at[idxd.at[b]], db, sem)

        def g_wait(b, i):
            sb, db, _, sem, _ = bufs[i]
            pltpu.make_async_copy(tab_ref.at[idxs.at[b]], sb, sem).wait()
            pltpu.make_async_copy(adt_ref.at[idxd.at[b]], db, sem).wait()

        def s_wait(b, i):
            _, _, ob, _, sem = bufs[i]
            pltpu.make_async_copy(ob, accum.at[idxd.at[b]], sem).wait()

        def compute_block(b, i):
            sb, db, ob, _, ssem = bufs[i]

            def edge(j, _):
                sa = sb[j, pl.ds(64, 16)]
                da = db[j, pl.ds(0, 16)]
                p = _leaky_exp(sa + da)
                for kk in range(4):
                    hk = sb[j, pl.ds(kk * 16, 16)]
                    ob[j, pl.ds(kk * 16, 16)] = hk * _bcast_pair(p, pidx[kk])
                ob[j, pl.ds(64, 16)] = p
                return 0

            lax.fori_loop(0, EB, edge, 0, unroll=4)
            pltpu.async_copy(ob, accum.at[idxd.at[b]], ssem, add=True)

        g_start(0, 0)

        def pair(g2, _):
            for i in range(2):
                b = 2 * g2 + i

                @pl.when(b < NB)
                def _():
                    g_wait(b, i)

                    @pl.when(b + 1 < NB)
                    def _():
                        g_start(b + 1, 1 - i)

                    @pl.when(b >= 2)
                    def _():
                        s_wait(b - 2, i)

                    compute_block(b, i)
            return 0

        lax.fori_loop(0, (NB + 1) // 2, pair, 0)
        s_wait(NB - 2, (NB - 2) % 2)
        s_wait(NB - 1, (NB - 1) % 2)
        plsc.subcore_barrier()
        _dump_accum(accum, out_ref, c, base)

    return k(tab, adt, esrc, edst)


def _edge_pass2(tab, as2, ad2, esrc, edst):
    mesh = plsc.VectorSubcoreMesh(core_axis_name="c", subcore_axis_name="s")

    @functools.partial(
        pl.kernel,
        out_type=jax.ShapeDtypeStruct((NC, NPAD, W2T), jnp.float32),
        mesh=mesh,
        compiler_params=_SC_PARAMS,
        scratch_types=[
            pltpu.VMEM((NB, EB), jnp.int32),
            pltpu.VMEM((NB, EB), jnp.int32),
            pltpu.VMEM((EB, 16), jnp.float32),
            pltpu.VMEM((EB, 16), jnp.float32),
            pltpu.VMEM((EB, W2T), jnp.float32),
            pltpu.VMEM((EB, W2T), jnp.float32),
            pltpu.VMEM((NPAD,), jnp.float32),
            pltpu.VMEM((NPAD,), jnp.float32),
            pltpu.VMEM((EB,), jnp.float32),
            pltpu.SemaphoreType.DMA,
            pltpu.SemaphoreType.DMA,
            pltpu.SemaphoreType.DMA,
            pltpu.SemaphoreType.DMA,
            pltpu.VMEM_SHARED((NPAD, W2T), jnp.float32),
        ],
    )
    def k(tab_ref, as2_ref, ad2_ref, esrc_ref, edst_ref, out_ref,
          idxs, idxd, srows0, srows1, orows0, orows1, asv, adv, pbuf,
          gsem0, gsem1, ssem0, ssem1, accum):
        c = lax.axis_index("c")
        s = lax.axis_index("s")
        wid = c * NS + s
        pltpu.sync_copy(esrc_ref.at[wid], idxs)
        pltpu.sync_copy(edst_ref.at[wid], idxd)
        pltpu.sync_copy(as2_ref, asv)
        pltpu.sync_copy(ad2_ref, adv)
        base = _zero_accum(orows0, accum, s, W2T)
        plsc.subcore_barrier()

        bufs = ((srows0, orows0, gsem0, ssem0), (srows1, orows1, gsem1, ssem1))

        def g_start(b, i):
            sb, _, sem, _ = bufs[i]
            pltpu.async_copy(tab_ref.at[idxs.at[b]], sb, sem)

        def g_wait(b, i):
            sb, _, sem, _ = bufs[i]
            pltpu.make_async_copy(tab_ref.at[idxs.at[b]], sb, sem).wait()

        def s_wait(b, i):
            _, ob, _, sem = bufs[i]
            pltpu.make_async_copy(ob, accum.at[idxd.at[b]], sem).wait()

        def compute_block(b, i):
            sb, ob, _, ssem = bufs[i]

            def grp(g, _):
                srcv = idxs[b, pl.ds(g * 16, 16)]
                dstv = idxd[b, pl.ds(g * 16, 16)]
                e = (plsc.load_gather(asv, [srcv]) +
                     plsc.load_gather(adv, [dstv]))
                pbuf[pl.ds(g * 16, 16)] = _leaky_exp(e)
                return 0

            lax.fori_loop(0, EB // 16, grp, 0, unroll=2)

            def edge(j, _):
                pv = plsc.load_gather(pbuf, [jnp.broadcast_to(j, (16,))])
                ob[j, pl.ds(0, 16)] = sb[j, pl.ds(0, 16)] * pv
                ob[j, pl.ds(16, 16)] = pv
                return 0

            lax.fori_loop(0, EB, edge, 0, unroll=4)
            pltpu.async_copy(ob, accum.at[idxd.at[b]], ssem, add=True)

        g_start(0, 0)

        def pair(g2, _):
            for i in range(2):
                b = 2 * g2 + i

                @pl.when(b < NB)
                def _():
                    g_wait(b, i)

                    @pl.when(b + 1 < NB)
                    def _():
                        g_start(b + 1, 1 - i)

                    @pl.when(b >= 2)
                    def _():
                        s_wait(b - 2, i)

                    compute_block(b, i)
            return 0

        lax.fori_loop(0, (NB + 1) // 2, pair, 0)
        s_wait(NB - 2, (NB - 2) % 2)
        s_wait(NB - 1, (NB - 1) % 2)
        plsc.subcore_barrier()
        _dump_accum(accum, out_ref, c, base)

    return k(tab, as2, ad2, esrc, edst)


def kernel(x, edge_index, W1, a1_src, a1_dst, b1, W2, a2_src, a2_dst, b2):
    eye8 = jnp.eye(8, dtype=jnp.float32)
    AMs = jnp.pad((a1_src[:, :, None] * eye8[:, None, :]).reshape(64, 8),
                  ((0, 0), (0, 8)))
    AMd = jnp.pad((a1_dst[:, :, None] * eye8[:, None, :]).reshape(64, 8),
                  ((0, 0), (0, 8)))
    R = jnp.pad(jnp.repeat(eye8, 8, axis=1), ((0, 8), (0, 0)))
    ones16 = jnp.ones((1, 16), dtype=jnp.float32)
    Ms2 = a2_src.reshape(16, 1) @ ones16
    Md2 = a2_dst.reshape(16, 1) @ ones16

    xp = jnp.pad(x, ((0, NPAD - N), (0, 0)))
    tab1, adt1 = _tc1(xp, W1, AMs, AMd)

    ei = jnp.pad(edge_index, ((0, 0), (0, NW * EPW - E)), constant_values=N)
    esrc = ei[0].reshape(NW, NB, EB)
    edst = ei[1].reshape(NW, NB, EB)

    acc1 = _edge_pass1(tab1, adt1, esrc, edst)
    tab2, as2b, ad2b = _tc2(acc1, b1.reshape(1, 64), W2, Ms2, Md2, R)
    acc2 = _edge_pass2(tab2, as2b[:, 0], ad2b[:, 0], esrc, edst)
    out = _tc3(acc2, b2.reshape(1, 16))
    return out[:N]


# speedup vs baseline: 95.5819x; 1.1433x over previous
import functools

import jax
import jax.numpy as jnp
import numpy as np
from jax import lax
from jax.experimental import pallas as pl
from jax.experimental.pallas import tpu as pltpu
from jax.experimental.pallas import tpu_sc as plsc

N = 10000
E = 320000
F_IN = 128

NPAD = 10240
NC, NS = 2, 16
NW = NC * NS
EB = 128
EPW = 10112
NB = EPW // EB
ROWS_PER_SUB = NPAD // NS
RCHUNK = 128
NRC = ROWS_PER_SUB // RCHUNK
W1T = 80
W2T = 32

_NBLK = 512
_GRID = NPAD // _NBLK

_SC_PARAMS = pltpu.CompilerParams(use_tc_tiling_on_sc=False,
                                  needs_layout_passes=False)


def _tc1_body(x_ref, w1_ref, ams_ref, amd_ref, tab_ref, adt_ref):
    h = jnp.dot(x_ref[...], w1_ref[...], preferred_element_type=jnp.float32)
    asb = jnp.dot(h, ams_ref[...], preferred_element_type=jnp.float32)
    adb = jnp.dot(h, amd_ref[...], preferred_element_type=jnp.float32)
    tab_ref[...] = jnp.concatenate([h, asb], axis=1)
    adt_ref[...] = adb


def _tc1(xp, W1, AMs, AMd):
    return pl.pallas_call(
        _tc1_body,
        grid=(_GRID,),
        in_specs=[
            pl.BlockSpec((_NBLK, F_IN), lambda i: (i, 0)),
            pl.BlockSpec((F_IN, 64), lambda i: (0, 0)),
            pl.BlockSpec((64, 16), lambda i: (0, 0)),
            pl.BlockSpec((64, 16), lambda i: (0, 0)),
        ],
        out_specs=[
            pl.BlockSpec((_NBLK, W1T), lambda i: (i, 0)),
            pl.BlockSpec((_NBLK, 16), lambda i: (i, 0)),
        ],
        out_shape=[
            jax.ShapeDtypeStruct((NPAD, W1T), jnp.float32),
            jax.ShapeDtypeStruct((NPAD, 16), jnp.float32),
        ],
    )(xp, W1, AMs, AMd)


def _tc2_body(acc_ref, b1_ref, w2_ref, ms_ref, md_ref, r_ref,
              tab_ref, as2_ref, ad2_ref):
    msg = acc_ref[0, :, :64] + acc_ref[1, :, :64]
    den16 = acc_ref[0, :, 64:80] + acc_ref[1, :, 64:80]
    den = jnp.dot(den16, r_ref[...], preferred_element_type=jnp.float32)
    hl = msg / (den + 1e-16) + b1_ref[...]
    hl = jnp.where(hl > 0, hl, jnp.exp(hl) - 1.0)
    h2 = jnp.dot(hl, w2_ref[...], preferred_element_type=jnp.float32)
    asb = jnp.dot(h2, ms_ref[...], preferred_element_type=jnp.float32)
    adb = jnp.dot(h2, md_ref[...], preferred_element_type=jnp.float32)
    tab_ref[...] = h2
    as2_ref[...] = asb
    ad2_ref[...] = adb


def _tc2(acc1, b1, W2, Ms2, Md2, R):
    return pl.pallas_call(
        _tc2_body,
        grid=(_GRID,),
        in_specs=[
            pl.BlockSpec((2, _NBLK, W1T), lambda i: (0, i, 0)),
            pl.BlockSpec((1, 64), lambda i: (0, 0)),
            pl.BlockSpec((64, 16), lambda i: (0, 0)),
            pl.BlockSpec((16, 16), lambda i: (0, 0)),
            pl.BlockSpec((16, 16), lambda i: (0, 0)),
            pl.BlockSpec((16, 64), lambda i: (0, 0)),
        ],
        out_specs=[
            pl.BlockSpec((_NBLK, 16), lambda i: (i, 0)),
            pl.BlockSpec((_NBLK, 16), lambda i: (i, 0)),
            pl.BlockSpec((_NBLK, 16), lambda i: (i, 0)),
        ],
        out_shape=[
            jax.ShapeDtypeStruct((NPAD, 16), jnp.float32),
            jax.ShapeDtypeStruct((NPAD, 16), jnp.float32),
            jax.ShapeDtypeStruct((NPAD, 16), jnp.float32),
        ],
    )(acc1, b1, W2, Ms2, Md2, R)


def _tc3_body(acc_ref, b2_ref, out_ref):
    msg = acc_ref[0, :, :16] + acc_ref[1, :, :16]
    den = acc_ref[0, :, 16:32] + acc_ref[1, :, 16:32]
    o = msg / (den + 1e-16) + b2_ref[...]
    m = jnp.max(o, axis=1, keepdims=True)
    z = o - m
    out_ref[...] = z - jnp.log(jnp.sum(jnp.exp(z), axis=1, keepdims=True))


def _tc3(acc2, b2):
    return pl.pallas_call(
        _tc3_body,
        grid=(_GRID,),
        in_specs=[
            pl.BlockSpec((2, _NBLK, W2T), lambda i: (0, i, 0)),
            pl.BlockSpec((1, 16), lambda i: (0, 0)),
        ],
        out_specs=pl.BlockSpec((_NBLK, 16), lambda i: (i, 0)),
        out_shape=jax.ShapeDtypeStruct((NPAD, 16), jnp.float32),
    )(acc2, b2)


_GDN = lax.GatherDimensionNumbers(
    offset_dims=(), collapsed_slice_dims=(0,), start_index_map=(0,))


def _pair_idx():
    io = lax.iota(jnp.int32, 16)
    hi = jnp.where(io >= 8, 1, 0)
    return [(2 * k + hi).reshape(16, 1) for k in range(4)]


def _bcast_pair(p, idx):
    return lax.gather(p, idx, _GDN, (1,),
                      mode=lax.GatherScatterMode.PROMISE_IN_BOUNDS)


def _leaky_exp(e):
    return jnp.exp(jnp.maximum(e, e * 0.2))


def _zero_accum(orows, accum, s, w):
    zv = jnp.zeros((16,), jnp.float32)

    def zrow(i, _):
        for kk in range(w // 16):
            orows[i, pl.ds(kk * 16, 16)] = zv
        return 0

    lax.fori_loop(0, EB, zrow, 0)
    base = s * ROWS_PER_SUB
    for jj in range(NRC):
        pltpu.sync_copy(orows, accum.at[pl.ds(base + jj * RCHUNK, RCHUNK)])
    return base


def _dump_accum(accum, out_ref, c, base):
    for jj in range(NRC):
        r0 = base + jj * RCHUNK
        pltpu.sync_copy(accum.at[pl.ds(r0, RCHUNK)],
                        out_ref.at[c, pl.ds(r0, RCHUNK)])


def _edge_pass1(tab, adt, esrc, edst):
    mesh = plsc.VectorSubcoreMesh(core_axis_name="c", subcore_axis_name="s")

    @functools.partial(
        pl.kernel,
        out_type=jax.ShapeDtypeStruct((NC, NPAD, W1T), jnp.float32),
        mesh=mesh,
        compiler_params=_SC_PARAMS,
        scratch_types=[
            pltpu.VMEM((NB, EB), jnp.int32),
            pltpu.VMEM((NB, EB), jnp.int32),
            pltpu.VMEM((EB, W1T), jnp.float32),
            pltpu.VMEM((EB, W1T), jnp.float32),
            pltpu.VMEM((EB, 16), jnp.float32),
            pltpu.VMEM((EB, 16), jnp.float32),
            pltpu.VMEM((EB, W1T), jnp.float32),
            pltpu.VMEM((EB, W1T), jnp.float32),
            pltpu.SemaphoreType.DMA,
            pltpu.SemaphoreType.DMA,
            pltpu.SemaphoreType.DMA,
            pltpu.SemaphoreType.DMA,
            pltpu.VMEM_SHARED((NPAD, W1T), jnp.float32),
        ],
    )
    def k(tab_ref, adt_ref, esrc_ref, edst_ref, out_ref,
          idxs, idxd, srows0, srows1, drows0, drows1, orows0, orows1,
          gsem0, gsem1, ssem0, ssem1, accum):
        c = lax.axis_index("c")
        s = lax.axis_index("s")
        wid = c * NS + s
        pltpu.sync_copy(esrc_ref.at[wid], idxs)
        pltpu.sync_copy(edst_ref.at[wid], idxd)
        base = _zero_accum(orows0, accum, s, W1T)
        plsc.subcore_barrier()

        bufs = ((srows0, drows0, orows0, gsem0, ssem0),
                (srows1, drows1, orows1, gsem1, ssem1))
        pidx = _pair_idx()

        def g_start(b, i):
            sb, db, _, sem, _ = bufs[i]
            pltpu.async_copy(tab_ref.at[idxs.at[b]], sb, sem)
            pltpu.async_copy(adt_ref.at[idxd.at[b]], db, sem)

        def g_wait(b, i):
            sb, db, _, sem, _ = bufs[i]
            pltpu.make_async_copy(tab_ref.at[idxs.at[b]], sb, sem).wait()
            pltpu.make_async_copy(adt_ref.at[idxd.at[b]], db, sem).wait()

        def s_wait(b, i):
            _, _, ob, _, sem = bufs[i]
            pltpu.make_async_copy(ob, accum.at[idxd.at[b]], sem).wait()

        def compute_block(b, i):
            sb, db, ob, _, ssem = bufs[i]

            def compute_p(j):
                sa = sb[j, pl.ds(64, 16)]
                da = db[j, pl.ds(0, 16)]
                return _leaky_exp(sa + da)

            def do_msg(j, p):
                for kk in range(4):
                    hk = sb[j, pl.ds(kk * 16, 16)]
                    ob[j, pl.ds(kk * 16, 16)] = hk * _bcast_pair(p, pidx[kk])
                ob[j, pl.ds(64, 16)] = p

            def edge(j, p_prev):
                p_new = compute_p(j)
                do_msg(j - 1, p_prev)
                return p_new

            p_last = lax.fori_loop(1, EB, edge, compute_p(0), unroll=4)
            do_msg(EB - 1, p_last)
            pltpu.async_copy(ob, accum.at[idxd.at[b]], ssem, add=True)

        g_start(0, 0)

        def pair(g2, _):
            for i in range(2):
                b = 2 * g2 + i

                @pl.when(b < NB)
                def _():
                    g_wait(b, i)

                    @pl.when(b + 1 < NB)
                    def _():
                        g_start(b + 1, 1 - i)

                    @pl.when(b >= 2)
                    def _():
                        s_wait(b - 2, i)

                    compute_block(b, i)
            return 0

        lax.fori_loop(0, (NB + 1) // 2, pair, 0)
        s_wait(NB - 2, (NB - 2) % 2)
        s_wait(NB - 1, (NB - 1) % 2)
        plsc.subcore_barrier()
        _dump_accum(accum, out_ref, c, base)

    return k(tab, adt, esrc, edst)


def _edge_pass2(tab, as2, ad2, esrc, edst):
    mesh = plsc.VectorSubcoreMesh(core_axis_name="c", subcore_axis_name="s")

    @functools.partial(
        pl.kernel,
        out_type=jax.ShapeDtypeStruct((NC, NPAD, W2T), jnp.float32),
        mesh=mesh,
        compiler_params=_SC_PARAMS,
        scratch_types=[
            pltpu.VMEM((NB, EB), jnp.int32),
            pltpu.VMEM((NB, EB), jnp.int32),
            pltpu.VMEM((EB, 16), jnp.float32),
            pltpu.VMEM((EB, 16), jnp.float32),
            pltpu.VMEM((EB, W2T), jnp.float32),
            pltpu.VMEM((EB, W2T), jnp.float32),
            pltpu.VMEM((NPAD,), jnp.float32),
            pltpu.VMEM((NPAD,), jnp.float32),
            pltpu.VMEM((EB,), jnp.float32),
            pltpu.SemaphoreType.DMA,
            pltpu.SemaphoreType.DMA,
            pltpu.SemaphoreType.DMA,
            pltpu.SemaphoreType.DMA,
            pltpu.VMEM_SHARED((NPAD, W2T), jnp.float32),
        ],
    )
    def k(tab_ref, as2_ref, ad2_ref, esrc_ref, edst_ref, out_ref,
          idxs, idxd, srows0, srows1, orows0, orows1, asv, adv, pbuf,
          gsem0, gsem1, ssem0, ssem1, accum):
        c = lax.axis_index("c")
        s = lax.axis_index("s")
        wid = c * NS + s
        pltpu.sync_copy(esrc_ref.at[wid], idxs)
        pltpu.sync_copy(edst_ref.at[wid], idxd)
        pltpu.sync_copy(as2_ref, asv)
        pltpu.sync_copy(ad2_ref, adv)
        base = _zero_accum(orows0, accum, s, W2T)
        plsc.subcore_barrier()

        bufs = ((srows0, orows0, gsem0, ssem0), (srows1, orows1, gsem1, ssem1))

        def g_start(b, i):
            sb, _, sem, _ = bufs[i]
            pltpu.async_copy(tab_ref.at[idxs.at[b]], sb, sem)

        def g_wait(b, i):
            sb, _, sem, _ = bufs[i]
            pltpu.make_async_copy(tab_ref.at[idxs.at[b]], sb, sem).wait()

        def s_wait(b, i):
            _, ob, _, sem = bufs[i]
            pltpu.make_async_copy(ob, accum.at[idxd.at[b]], sem).wait()

        def compute_block(b, i):
            sb, ob, _, ssem = bufs[i]

            def grp(g, _):
                srcv = idxs[b, pl.ds(g * 16, 16)]
                dstv = idxd[b, pl.ds(g * 16, 16)]
                e = (plsc.load_gather(asv, [srcv]) +
                     plsc.load_gather(adv, [dstv]))
                pbuf[pl.ds(g * 16, 16)] = _leaky_exp(e)
                return 0

            lax.fori_loop(0, EB // 16, grp, 0, unroll=2)

            def edge(j, _):
                pv = plsc.load_gather(pbuf, [jnp.broadcast_to(j, (16,))])
                ob[j, pl.ds(0, 16)] = sb[j, pl.ds(0, 16)] * pv
                ob[j, pl.ds(16, 16)] = pv
                return 0

            lax.fori_loop(0, EB, edge, 0, unroll=4)
            pltpu.async_copy(ob, accum.at[idxd.at[b]], ssem, add=True)

        g_start(0, 0)

        def pair(g2, _):
            for i in range(2):
                b = 2 * g2 + i

                @pl.when(b < NB)
                def _():
                    g_wait(b, i)

                    @pl.when(b + 1 < NB)
                    def _():
                        g_start(b + 1, 1 - i)

                    @pl.when(b >= 2)
                    def _():
                        s_wait(b - 2, i)

                    compute_block(b, i)
            return 0

        lax.fori_loop(0, (NB + 1) // 2, pair, 0)
        s_wait(NB - 2, (NB - 2) % 2)
        s_wait(NB - 1, (NB - 1) % 2)
        plsc.subcore_barrier()
        _dump_accum(accum, out_ref, c, base)

    return k(tab, as2, ad2, esrc, edst)


def kernel(x, edge_index, W1, a1_src, a1_dst, b1, W2, a2_src, a2_dst, b2):
    eye8 = jnp.eye(8, dtype=jnp.float32)
    AMs = jnp.pad((a1_src[:, :, None] * eye8[:, None, :]).reshape(64, 8),
                  ((0, 0), (0, 8)))
    AMd = jnp.pad((a1_dst[:, :, None] * eye8[:, None, :]).reshape(64, 8),
                  ((0, 0), (0, 8)))
    R = jnp.pad(jnp.repeat(eye8, 8, axis=1), ((0, 8), (0, 0)))
    ones16 = jnp.ones((1, 16), dtype=jnp.float32)
    Ms2 = a2_src.reshape(16, 1) @ ones16
    Md2 = a2_dst.reshape(16, 1) @ ones16

    xp = jnp.pad(x, ((0, NPAD - N), (0, 0)))
    tab1, adt1 = _tc1(xp, W1, AMs, AMd)

    ei = jnp.pad(edge_index, ((0, 0), (0, NW * EPW - E)), constant_values=N)
    esrc = ei[0].reshape(NW, NB, EB)
    edst = ei[1].reshape(NW, NB, EB)

    acc1 = _edge_pass1(tab1, adt1, esrc, edst)
    tab2, as2b, ad2b = _tc2(acc1, b1.reshape(1, 64), W2, Ms2, Md2, R)
    acc2 = _edge_pass2(tab2, as2b[:, 0], ad2b[:, 0], esrc, edst)
    out = _tc3(acc2, b2.reshape(1, 16))
    return out[:N]


# speedup vs baseline: 95.9803x; 1.0042x over previous
import functools

import jax
import jax.numpy as jnp
import numpy as np
from jax import lax
from jax.experimental import pallas as pl
from jax.experimental.pallas import tpu as pltpu
from jax.experimental.pallas import tpu_sc as plsc

N = 10000
E = 320000
F_IN = 128

NPAD = 10240
NC, NS = 2, 16
NW = NC * NS
EB = 128
EPW = 10112
NB = EPW // EB
ROWS_PER_SUB = NPAD // NS
RCHUNK = 128
NRC = ROWS_PER_SUB // RCHUNK
W1T = 80
W2T = 32

_NBLK = 512
_GRID = NPAD // _NBLK

_SC_PARAMS = pltpu.CompilerParams(use_tc_tiling_on_sc=False,
                                  needs_layout_passes=False)


def _tc1_body(x_ref, w1_ref, ams_ref, amd_ref, tab_ref, adt_ref):
    h = jnp.dot(x_ref[...], w1_ref[...], preferred_element_type=jnp.float32)
    asb = jnp.dot(h, ams_ref[...], preferred_element_type=jnp.float32)
    adb = jnp.dot(h, amd_ref[...], preferred_element_type=jnp.float32)
    tab_ref[...] = jnp.concatenate([h, asb], axis=1)
    adt_ref[...] = adb


def _tc1(xp, W1, AMs, AMd):
    return pl.pallas_call(
        _tc1_body,
        grid=(_GRID,),
        in_specs=[
            pl.BlockSpec((_NBLK, F_IN), lambda i: (i, 0)),
            pl.BlockSpec((F_IN, 64), lambda i: (0, 0)),
            pl.BlockSpec((64, 16), lambda i: (0, 0)),
            pl.BlockSpec((64, 16), lambda i: (0, 0)),
        ],
        out_specs=[
            pl.BlockSpec((_NBLK, W1T), lambda i: (i, 0)),
            pl.BlockSpec((_NBLK, 16), lambda i: (i, 0)),
        ],
        out_shape=[
            jax.ShapeDtypeStruct((NPAD, W1T), jnp.float32),
            jax.ShapeDtypeStruct((NPAD, 16), jnp.float32),
        ],
    )(xp, W1, AMs, AMd)


def _tc2_body(acc_ref, b1_ref, w2_ref, ms_ref, md_ref, r_ref,
              tab_ref, as2_ref, ad2_ref):
    msg = acc_ref[0, :, :64] + acc_ref[1, :, :64]
    den16 = acc_ref[0, :, 64:80] + acc_ref[1, :, 64:80]
    den = jnp.dot(den16, r_ref[...], preferred_element_type=jnp.float32)
    hl = msg / (den + 1e-16) + b1_ref[...]
    hl = jnp.where(hl > 0, hl, jnp.exp(hl) - 1.0)
    h2 = jnp.dot(hl, w2_ref[...], preferred_element_type=jnp.float32)
    asb = jnp.dot(h2, ms_ref[...], preferred_element_type=jnp.float32)
    adb = jnp.dot(h2, md_ref[...], preferred_element_type=jnp.float32)
    tab_ref[...] = h2
    as2_ref[...] = asb
    ad2_ref[...] = adb


def _tc2(acc1, b1, W2, Ms2, Md2, R):
    return pl.pallas_call(
        _tc2_body,
        grid=(_GRID,),
        in_specs=[
            pl.BlockSpec((2, _NBLK, W1T), lambda i: (0, i, 0)),
            pl.BlockSpec((1, 64), lambda i: (0, 0)),
            pl.BlockSpec((64, 16), lambda i: (0, 0)),
            pl.BlockSpec((16, 16), lambda i: (0, 0)),
            pl.BlockSpec((16, 16), lambda i: (0, 0)),
            pl.BlockSpec((16, 64), lambda i: (0, 0)),
        ],
        out_specs=[
            pl.BlockSpec((_NBLK, 16), lambda i: (i, 0)),
            pl.BlockSpec((_NBLK, 16), lambda i: (i, 0)),
            pl.BlockSpec((_NBLK, 16), lambda i: (i, 0)),
        ],
        out_shape=[
            jax.ShapeDtypeStruct((NPAD, 16), jnp.float32),
            jax.ShapeDtypeStruct((NPAD, 16), jnp.float32),
            jax.ShapeDtypeStruct((NPAD, 16), jnp.float32),
        ],
    )(acc1, b1, W2, Ms2, Md2, R)


def _tc3_body(acc_ref, b2_ref, out_ref):
    msg = acc_ref[0, :, :16] + acc_ref[1, :, :16]
    den = acc_ref[0, :, 16:32] + acc_ref[1, :, 16:32]
    o = msg / (den + 1e-16) + b2_ref[...]
    m = jnp.max(o, axis=1, keepdims=True)
    z = o - m
    out_ref[...] = z - jnp.log(jnp.sum(jnp.exp(z), axis=1, keepdims=True))


def _tc3(acc2, b2):
    return pl.pallas_call(
        _tc3_body,
        grid=(_GRID,),
        in_specs=[
            pl.BlockSpec((2, _NBLK, W2T), lambda i: (0, i, 0)),
            pl.BlockSpec((1, 16), lambda i: (0, 0)),
        ],
        out_specs=pl.BlockSpec((_NBLK, 16), lambda i: (i, 0)),
        out_shape=jax.ShapeDtypeStruct((NPAD, 16), jnp.float32),
    )(acc2, b2)


_GDN = lax.GatherDimensionNumbers(
    offset_dims=(), collapsed_slice_dims=(0,), start_index_map=(0,))


def _pair_idx():
    io = lax.iota(jnp.int32, 16)
    hi = jnp.where(io >= 8, 1, 0)
    return [(2 * k + hi).reshape(16, 1) for k in range(4)]


def _bcast_pair(p, idx):
    return lax.gather(p, idx, _GDN, (1,),
                      mode=lax.GatherScatterMode.PROMISE_IN_BOUNDS)


def _leaky_exp(e):
    return jnp.exp(jnp.maximum(e, e * 0.2))


def _zero_accum(orows, accum, s, w):
    zv = jnp.zeros((16,), jnp.float32)

    def zrow(i, _):
        for kk in range(w // 16):
            orows[i, pl.ds(kk * 16, 16)] = zv
        return 0

    lax.fori_loop(0, EB, zrow, 0)
    base = s * ROWS_PER_SUB
    for jj in range(NRC):
        pltpu.sync_copy(orows, accum.at[pl.ds(base + jj * RCHUNK, RCHUNK)])
    return base


def _dump_accum(accum, out_ref, c, base):
    for jj in range(NRC):
        r0 = base + jj * RCHUNK
        pltpu.sync_copy(accum.at[pl.ds(r0, RCHUNK)],
                        out_ref.at[c, pl.ds(r0, RCHUNK)])


def _edge_pass1(tab, adt, esrc, edst):
    mesh = plsc.VectorSubcoreMesh(core_axis_name="c", subcore_axis_name="s")

    @functools.partial(
        pl.kernel,
        out_type=jax.ShapeDtypeStruct((NC, NPAD, W1T), jnp.float32),
        mesh=mesh,
        compiler_params=_SC_PARAMS,
        scratch_types=[
            pltpu.VMEM((NB, EB), jnp.int32),
            pltpu.VMEM((NB, EB), jnp.int32),
            pltpu.VMEM((EB, W1T), jnp.float32),
            pltpu.VMEM((EB, W1T), jnp.float32),
            pltpu.VMEM((EB, 16), jnp.float32),
            pltpu.VMEM((EB, 16), jnp.float32),
            pltpu.VMEM((EB, W1T), jnp.float32),
            pltpu.VMEM((EB, W1T), jnp.float32),
            pltpu.SemaphoreType.DMA,
            pltpu.SemaphoreType.DMA,
            pltpu.SemaphoreType.DMA,
            pltpu.SemaphoreType.DMA,
            pltpu.VMEM_SHARED((NPAD, W1T), jnp.float32),
        ],
    )
    def k(tab_ref, adt_ref, esrc_ref, edst_ref, out_ref,
          idxs, idxd, srows0, srows1, drows0, drows1, orows0, orows1,
          gsem0, gsem1, ssem0, ssem1, accum):
        c = lax.axis_index("c")
        s = lax.axis_index("s")
        wid = c * NS + s
        pltpu.sync_copy(esrc_ref.at[wid], idxs)
        pltpu.sync_copy(edst_ref.at[wid], idxd)
        base = _zero_accum(orows0, accum, s, W1T)
        plsc.subcore_barrier()

        bufs = ((srows0, drows0, orows0, gsem0, ssem0),
                (srows1, drows1, orows1, gsem1, ssem1))
        pidx = _pair_idx()

        def g_start(b, i):
            sb, db, _, sem, _ = bufs[i]
            pltpu.async_copy(tab_ref.at[idxs.at[b]], sb, sem)
            pltpu.async_copy(adt_ref.at[idxd.at[b]], db, sem)

        def g_wait(b, i):
            sb, db, _, sem, _ = bufs[i]
            pltpu.make_async_copy(tab_ref.at[idxs.at[b]], sb, sem).wait()
            pltpu.make_async_copy(adt_ref.at[idxd.at[b]], db, sem).wait()

        def s_wait(b, i):
            _, _, ob, _, sem = bufs[i]
            pltpu.make_async_copy(ob, accum.at[idxd.at[b]], sem).wait()

        def compute_block(b, i):
            sb, db, ob, _, ssem = bufs[i]

            def compute_p(j):
                sa = sb[j, pl.ds(64, 16)]
                da = db[j, pl.ds(0, 16)]
                return _leaky_exp(sa + da)

            def do_msg(j, p):
                for kk in range(4):
                    hk = sb[j, pl.ds(kk * 16, 16)]
                    ob[j, pl.ds(kk * 16, 16)] = hk * _bcast_pair(p, pidx[kk])
                ob[j, pl.ds(64, 16)] = p

            def edge(j, carry):
                p1, p2 = carry
                p_new = compute_p(j)
                do_msg(j - 2, p2)
                return (p_new, p1)

            pl_, pl2 = lax.fori_loop(2, EB, edge,
                                     (compute_p(1), compute_p(0)), unroll=8)
            do_msg(EB - 2, pl2)
            do_msg(EB - 1, pl_)
            pltpu.async_copy(ob, accum.at[idxd.at[b]], ssem, add=True)

        g_start(0, 0)

        def pair(g2, _):
            for i in range(2):
                b = 2 * g2 + i

                @pl.when(b < NB)
                def _():
                    g_wait(b, i)

                    @pl.when(b + 1 < NB)
                    def _():
                        g_start(b + 1, 1 - i)

                    @pl.when(b >= 2)
                    def _():
                        s_wait(b - 2, i)

                    compute_block(b, i)
            return 0

        lax.fori_loop(0, (NB + 1) // 2, pair, 0)
        s_wait(NB - 2, (NB - 2) % 2)
        s_wait(NB - 1, (NB - 1) % 2)
        plsc.subcore_barrier()
        _dump_accum(accum, out_ref, c, base)

    return k(tab, adt, esrc, edst)


def _edge_pass2(tab, as2, ad2, esrc, edst):
    mesh = plsc.VectorSubcoreMesh(core_axis_name="c", subcore_axis_name="s")

    @functools.partial(
        pl.kernel,
        out_type=jax.ShapeDtypeStruct((NC, NPAD, W2T), jnp.float32),
        mesh=mesh,
        compiler_params=_SC_PARAMS,
        scratch_types=[
            pltpu.VMEM((NB, EB), jnp.int32),
            pltpu.VMEM((NB, EB), jnp.int32),
            pltpu.VMEM((EB, 16), jnp.float32),
            pltpu.VMEM((EB, 16), jnp.float32),
            pltpu.VMEM((EB, W2T), jnp.float32),
            pltpu.VMEM((EB, W2T), jnp.float32),
            pltpu.VMEM((NPAD,), jnp.float32),
            pltpu.VMEM((NPAD,), jnp.float32),
            pltpu.VMEM((EB,), jnp.float32),
            pltpu.SemaphoreType.DMA,
            pltpu.SemaphoreType.DMA,
            pltpu.SemaphoreType.DMA,
            pltpu.SemaphoreType.DMA,
            pltpu.VMEM_SHARED((NPAD, W2T), jnp.float32),
        ],
    )
    def k(tab_ref, as2_ref, ad2_ref, esrc_ref, edst_ref, out_ref,
          idxs, idxd, srows0, srows1, orows0, orows1, asv, adv, pbuf,
          gsem0, gsem1, ssem0, ssem1, accum):
        c = lax.axis_index("c")
        s = lax.axis_index("s")
        wid = c * NS + s
        pltpu.sync_copy(esrc_ref.at[wid], idxs)
        pltpu.sync_copy(edst_ref.at[wid], idxd)
        pltpu.sync_copy(as2_ref, asv)
        pltpu.sync_copy(ad2_ref, adv)
        base = _zero_accum(orows0, accum, s, W2T)
        plsc.subcore_barrier()

        bufs = ((srows0, orows0, gsem0, ssem0), (srows1, orows1, gsem1, ssem1))

        def g_start(b, i):
            sb, _, sem, _ = bufs[i]
            pltpu.async_copy(tab_ref.at[idxs.at[b]], sb, sem)

        def g_wait(b, i):
            sb, _, sem, _ = bufs[i]
            pltpu.make_async_copy(tab_ref.at[idxs.at[b]], sb, sem).wait()

        def s_wait(b, i):
            _, ob, _, sem = bufs[i]
            pltpu.make_async_copy(ob, accum.at[idxd.at[b]], sem).wait()

        def compute_block(b, i):
            sb, ob, _, ssem = bufs[i]

            def grp(g, _):
                srcv = idxs[b, pl.ds(g * 16, 16)]
                dstv = idxd[b, pl.ds(g * 16, 16)]
                e = (plsc.load_gather(asv, [srcv]) +
                     plsc.load_gather(adv, [dstv]))
                pbuf[pl.ds(g * 16, 16)] = _leaky_exp(e)
                return 0

            lax.fori_loop(0, EB // 16, grp, 0, unroll=2)

            def edge(j, _):
                pv = plsc.load_gather(pbuf, [jnp.broadcast_to(j, (16,))])
                ob[j, pl.ds(0, 16)] = sb[j, pl.ds(0, 16)] * pv
                ob[j, pl.ds(16, 16)] = pv
                return 0

            lax.fori_loop(0, EB, edge, 0, unroll=4)
            pltpu.async_copy(ob, accum.at[idxd.at[b]], ssem, add=True)

        g_start(0, 0)

        def pair(g2, _):
            for i in range(2):
                b = 2 * g2 + i

                @pl.when(b < NB)
                def _():
                    g_wait(b, i)

                    @pl.when(b + 1 < NB)
                    def _():
                        g_start(b + 1, 1 - i)

                    @pl.when(b >= 2)
                    def _():
                        s_wait(b - 2, i)

                    compute_block(b, i)
            return 0

        lax.fori_loop(0, (NB + 1) // 2, pair, 0)
        s_wait(NB - 2, (NB - 2) % 2)
        s_wait(NB - 1, (NB - 1) % 2)
        plsc.subcore_barrier()
        _dump_accum(accum, out_ref, c, base)

    return k(tab, as2, ad2, esrc, edst)


def kernel(x, edge_index, W1, a1_src, a1_dst, b1, W2, a2_src, a2_dst, b2):
    eye8 = jnp.eye(8, dtype=jnp.float32)
    AMs = jnp.pad((a1_src[:, :, None] * eye8[:, None, :]).reshape(64, 8),
                  ((0, 0), (0, 8)))
    AMd = jnp.pad((a1_dst[:, :, None] * eye8[:, None, :]).reshape(64, 8),
                  ((0, 0), (0, 8)))
    R = jnp.pad(jnp.repeat(eye8, 8, axis=1), ((0, 8), (0, 0)))
    ones16 = jnp.ones((1, 16), dtype=jnp.float32)
    Ms2 = a2_src.reshape(16, 1) @ ones16
    Md2 = a2_dst.reshape(16, 1) @ ones16

    xp = jnp.pad(x, ((0, NPAD - N), (0, 0)))
    tab1, adt1 = _tc1(xp, W1, AMs, AMd)

    ei = jnp.pad(edge_index, ((0, 0), (0, NW * EPW - E)), constant_values=N)
    esrc = ei[0].reshape(NW, NB, EB)
    edst = ei[1].reshape(NW, NB, EB)

    acc1 = _edge_pass1(tab1, adt1, esrc, edst)
    tab2, as2b, ad2b = _tc2(acc1, b1.reshape(1, 64), W2, Ms2, Md2, R)
    acc2 = _edge_pass2(tab2, as2b[:, 0], ad2b[:, 0], esrc, edst)
    out = _tc3(acc2, b2.reshape(1, 16))
    return out[:N]


# speedup vs baseline: 114.1394x; 1.1892x over previous
import functools

import jax
import jax.numpy as jnp
import numpy as np
from jax import lax
from jax.experimental import pallas as pl
from jax.experimental.pallas import tpu as pltpu
from jax.experimental.pallas import tpu_sc as plsc

N = 10000
E = 320000
F_IN = 128

NPAD = 10240
NC, NS = 2, 16
NW = NC * NS
EB = 128
B1C0, B1C1 = 97, 60
B2C0, B2C1 = 83, 74
NBT = B1C0 + B1C1
BMAX = max(B1C0, B1C1, B2C0, B2C1)
assert B2C0 + B2C1 == NBT
TBP = NS * NBT
ROWS_PER_SUB = NPAD // NS
RCHUNK = 128
NRC = ROWS_PER_SUB // RCHUNK
W1T = 80
W2T = 32

_NBLK = 512
_GRID = NPAD // _NBLK

_SC_PARAMS = pltpu.CompilerParams(use_tc_tiling_on_sc=False,
                                  needs_layout_passes=False)


def _tc1_body(x_ref, w1_ref, ams_ref, amd_ref, tab_ref, adt_ref):
    h = jnp.dot(x_ref[...], w1_ref[...], preferred_element_type=jnp.float32)
    asb = jnp.dot(h, ams_ref[...], preferred_element_type=jnp.float32)
    adb = jnp.dot(h, amd_ref[...], preferred_element_type=jnp.float32)
    tab_ref[...] = jnp.concatenate([h, asb], axis=1)
    adt_ref[...] = adb


def _tc1(xp, W1, AMs, AMd):
    return pl.pallas_call(
        _tc1_body,
        grid=(_GRID,),
        in_specs=[
            pl.BlockSpec((_NBLK, F_IN), lambda i: (i, 0)),
            pl.BlockSpec((F_IN, 64), lambda i: (0, 0)),
            pl.BlockSpec((64, 16), lambda i: (0, 0)),
            pl.BlockSpec((64, 16), lambda i: (0, 0)),
        ],
        out_specs=[
            pl.BlockSpec((_NBLK, W1T), lambda i: (i, 0)),
            pl.BlockSpec((_NBLK, 16), lambda i: (i, 0)),
        ],
        out_shape=[
            jax.ShapeDtypeStruct((NPAD, W1T), jnp.float32),
            jax.ShapeDtypeStruct((NPAD, 16), jnp.float32),
        ],
    )(xp, W1, AMs, AMd)


def _tc2_body(acc_ref, b1_ref, w2_ref, ms_ref, md_ref, r_ref,
              tab_ref, as2_ref, ad2_ref):
    msg = acc_ref[0, :, :64] + acc_ref[1, :, :64]
    den16 = acc_ref[0, :, 64:80] + acc_ref[1, :, 64:80]
    den = jnp.dot(den16, r_ref[...], preferred_element_type=jnp.float32)
    hl = msg / (den + 1e-16) + b1_ref[...]
    hl = jnp.where(hl > 0, hl, jnp.exp(hl) - 1.0)
    h2 = jnp.dot(hl, w2_ref[...], preferred_element_type=jnp.float32)
    asb = jnp.dot(h2, ms_ref[...], preferred_element_type=jnp.float32)
    adb = jnp.dot(h2, md_ref[...], preferred_element_type=jnp.float32)
    tab_ref[...] = h2
    as2_ref[...] = asb
    ad2_ref[...] = adb


def _tc2(acc1, b1, W2, Ms2, Md2, R):
    return pl.pallas_call(
        _tc2_body,
        grid=(_GRID,),
        in_specs=[
            pl.BlockSpec((2, _NBLK, W1T), lambda i: (0, i, 0)),
            pl.BlockSpec((1, 64), lambda i: (0, 0)),
            pl.BlockSpec((64, 16), lambda i: (0, 0)),
            pl.BlockSpec((16, 16), lambda i: (0, 0)),
            pl.BlockSpec((16, 16), lambda i: (0, 0)),
            pl.BlockSpec((16, 64), lambda i: (0, 0)),
        ],
        out_specs=[
            pl.BlockSpec((_NBLK, 16), lambda i: (i, 0)),
            pl.BlockSpec((_NBLK, 16), lambda i: (i, 0)),
            pl.BlockSpec((_NBLK, 16), lambda i: (i, 0)),
        ],
        out_shape=[
            jax.ShapeDtypeStruct((NPAD, 16), jnp.float32),
            jax.ShapeDtypeStruct((NPAD, 16), jnp.float32),
            jax.ShapeDtypeStruct((NPAD, 16), jnp.float32),
        ],
    )(acc1, b1, W2, Ms2, Md2, R)


def _tc3_body(acc_ref, b2_ref, out_ref):
    msg = acc_ref[0, :, :16] + acc_ref[1, :, :16]
    den = acc_ref[0, :, 16:32] + acc_ref[1, :, 16:32]
    o = msg / (den + 1e-16) + b2_ref[...]
    m = jnp.max(o, axis=1, keepdims=True)
    z = o - m
    out_ref[...] = z - jnp.log(jnp.sum(jnp.exp(z), axis=1, keepdims=True))


def _tc3(acc2, b2):
    return pl.pallas_call(
        _tc3_body,
        grid=(_GRID,),
        in_specs=[
            pl.BlockSpec((2, _NBLK, W2T), lambda i: (0, i, 0)),
            pl.BlockSpec((1, 16), lambda i: (0, 0)),
        ],
        out_specs=pl.BlockSpec((_NBLK, 16), lambda i: (i, 0)),
        out_shape=jax.ShapeDtypeStruct((NPAD, 16), jnp.float32),
    )(acc2, b2)


_GDN = lax.GatherDimensionNumbers(
    offset_dims=(), collapsed_slice_dims=(0,), start_index_map=(0,))


def _pair_idx():
    io = lax.iota(jnp.int32, 16)
    hi = jnp.where(io >= 8, 1, 0)
    return [(2 * k + hi).reshape(16, 1) for k in range(4)]


def _bcast_pair(p, idx):
    return lax.gather(p, idx, _GDN, (1,),
                      mode=lax.GatherScatterMode.PROMISE_IN_BOUNDS)


def _leaky_exp(e):
    return jnp.exp(jnp.maximum(e, e * 0.2))


def _zero_accum(orows, accum, s, w):
    zv = jnp.zeros((16,), jnp.float32)

    def zrow(i, _):
        for kk in range(w // 16):
            orows[i, pl.ds(kk * 16, 16)] = zv
        return 0

    lax.fori_loop(0, EB, zrow, 0)
    base = s * ROWS_PER_SUB
    for jj in range(NRC):
        pltpu.sync_copy(orows, accum.at[pl.ds(base + jj * RCHUNK, RCHUNK)])
    return base


def _dump_accum(accum, out_ref, c, base):
    for jj in range(NRC):
        r0 = base + jj * RCHUNK
        pltpu.sync_copy(accum.at[pl.ds(r0, RCHUNK)],
                        out_ref.at[c, pl.ds(r0, RCHUNK)])


def _edge_pass1(tab, adt, esrc, edst):
    mesh = plsc.VectorSubcoreMesh(core_axis_name="c", subcore_axis_name="s")

    @functools.partial(
        pl.kernel,
        out_type=jax.ShapeDtypeStruct((NC, NPAD, W1T), jnp.float32),
        mesh=mesh,
        compiler_params=_SC_PARAMS,
        scratch_types=[
            pltpu.VMEM((BMAX, EB), jnp.int32),
            pltpu.VMEM((BMAX, EB), jnp.int32),
            pltpu.VMEM((EB, W1T), jnp.float32),
            pltpu.VMEM((EB, W1T), jnp.float32),
            pltpu.VMEM((EB, 16), jnp.float32),
            pltpu.VMEM((EB, 16), jnp.float32),
            pltpu.VMEM((EB, W1T), jnp.float32),
            pltpu.VMEM((EB, W1T), jnp.float32),
            pltpu.SemaphoreType.DMA,
            pltpu.SemaphoreType.DMA,
            pltpu.SemaphoreType.DMA,
            pltpu.SemaphoreType.DMA,
            pltpu.VMEM_SHARED((NPAD, W1T), jnp.float32),
        ],
    )
    def k(tab_ref, adt_ref, esrc_ref, edst_ref, out_ref,
          idxs, idxd, srows0, srows1, drows0, drows1, orows0, orows1,
          gsem0, gsem1, ssem0, ssem1, accum):
        c = lax.axis_index("c")
        s = lax.axis_index("s")
        base = _zero_accum(orows0, accum, s, W1T)
        plsc.subcore_barrier()

        bufs = ((srows0, drows0, orows0, gsem0, ssem0),
                (srows1, drows1, orows1, gsem1, ssem1))
        pidx = _pair_idx()

        def g_start(b, i):
            sb, db, _, sem, _ = bufs[i]
            pltpu.async_copy(tab_ref.at[idxs.at[b]], sb, sem)
            pltpu.async_copy(adt_ref.at[idxd.at[b]], db, sem)

        def g_wait(b, i):
            sb, db, _, sem, _ = bufs[i]
            pltpu.make_async_copy(tab_ref.at[idxs.at[b]], sb, sem).wait()
            pltpu.make_async_copy(adt_ref.at[idxd.at[b]], db, sem).wait()

        def s_wait(b, i):
            _, _, ob, _, sem = bufs[i]
            pltpu.make_async_copy(ob, accum.at[idxd.at[b]], sem).wait()

        def compute_block(b, i):
            sb, db, ob, _, ssem = bufs[i]

            def compute_p(j):
                sa = sb[j, pl.ds(64, 16)]
                da = db[j, pl.ds(0, 16)]
                return _leaky_exp(sa + da)

            def do_msg(j, p):
                for kk in range(4):
                    hk = sb[j, pl.ds(kk * 16, 16)]
                    ob[j, pl.ds(kk * 16, 16)] = hk * _bcast_pair(p, pidx[kk])
                ob[j, pl.ds(64, 16)] = p

            def edge(j, carry):
                p1, p2 = carry
                p_new = compute_p(j)
                do_msg(j - 2, p2)
                return (p_new, p1)

            pl_, pl2 = lax.fori_loop(2, EB, edge,
                                     (compute_p(1), compute_p(0)), unroll=8)
            do_msg(EB - 2, pl2)
            do_msg(EB - 1, pl_)
            pltpu.async_copy(ob, accum.at[idxd.at[b]], ssem, add=True)

        def run(cnt, start):
            pltpu.sync_copy(esrc_ref.at[pl.ds(start, cnt)],
                            idxs.at[pl.ds(0, cnt)])
            pltpu.sync_copy(edst_ref.at[pl.ds(start, cnt)],
                            idxd.at[pl.ds(0, cnt)])
            g_start(0, 0)

            def pair(g2, _):
                for i in range(2):
                    b = 2 * g2 + i

                    @pl.when(b < cnt)
                    def _():
                        g_wait(b, i)

                        @pl.when(b + 1 < cnt)
                        def _():
                            g_start(b + 1, 1 - i)

                        @pl.when(b >= 2)
                        def _():
                            s_wait(b - 2, i)

                        compute_block(b, i)
                return 0

            lax.fori_loop(0, (cnt + 1) // 2, pair, 0)
            s_wait(cnt - 2, (cnt - 2) % 2)
            s_wait(cnt - 1, (cnt - 1) % 2)

        @pl.when(c == 0)
        def _():
            run(B1C0, s * B1C0)

        @pl.when(c == 1)
        def _():
            run(B1C1, NS * B1C0 + s * B1C1)

        plsc.subcore_barrier()
        _dump_accum(accum, out_ref, c, base)

    return k(tab, adt, esrc, edst)


def _edge_pass2(tab, as2, ad2, esrc, edst):
    mesh = plsc.VectorSubcoreMesh(core_axis_name="c", subcore_axis_name="s")

    @functools.partial(
        pl.kernel,
        out_type=jax.ShapeDtypeStruct((NC, NPAD, W2T), jnp.float32),
        mesh=mesh,
        compiler_params=_SC_PARAMS,
        scratch_types=[
            pltpu.VMEM((BMAX, EB), jnp.int32),
            pltpu.VMEM((BMAX, EB), jnp.int32),
            pltpu.VMEM((EB, 16), jnp.float32),
            pltpu.VMEM((EB, 16), jnp.float32),
            pltpu.VMEM((EB, W2T), jnp.float32),
            pltpu.VMEM((EB, W2T), jnp.float32),
            pltpu.VMEM((NPAD,), jnp.float32),
            pltpu.VMEM((NPAD,), jnp.float32),
            pltpu.VMEM((EB,), jnp.float32),
            pltpu.SemaphoreType.DMA,
            pltpu.SemaphoreType.DMA,
            pltpu.SemaphoreType.DMA,
            pltpu.SemaphoreType.DMA,
            pltpu.VMEM_SHARED((NPAD, W2T), jnp.float32),
        ],
    )
    def k(tab_ref, as2_ref, ad2_ref, esrc_ref, edst_ref, out_ref,
          idxs, idxd, srows0, srows1, orows0, orows1, asv, adv, pbuf,
          gsem0, gsem1, ssem0, ssem1, accum):
        c = lax.axis_index("c")
        s = lax.axis_index("s")
        pltpu.sync_copy(as2_ref, asv)
        pltpu.sync_copy(ad2_ref, adv)
        base = _zero_accum(orows0, accum, s, W2T)
        plsc.subcore_barrier()

        bufs = ((srows0, orows0, gsem0, ssem0), (srows1, orows1, gsem1, ssem1))

        def g_start(b, i):
            sb, _, sem, _ = bufs[i]
            pltpu.async_copy(tab_ref.at[idxs.at[b]], sb, sem)

        def g_wait(b, i):
            sb, _, sem, _ = bufs[i]
            pltpu.make_async_copy(tab_ref.at[idxs.at[b]], sb, sem).wait()

        def s_wait(b, i):
            _, ob, _, sem = bufs[i]
            pltpu.make_async_copy(ob, accum.at[idxd.at[b]], sem).wait()

        def compute_block(b, i):
            sb, ob, _, ssem = bufs[i]

            def grp(g, _):
                srcv = idxs[b, pl.ds(g * 16, 16)]
                dstv = idxd[b, pl.ds(g * 16, 16)]
                e = (plsc.load_gather(asv, [srcv]) +
                     plsc.load_gather(adv, [dstv]))
                pbuf[pl.ds(g * 16, 16)] = _leaky_exp(e)
                return 0

            lax.fori_loop(0, EB // 16, grp, 0, unroll=2)

            def edge(j, _):
                pv = plsc.load_gather(pbuf, [jnp.broadcast_to(j, (16,))])
                ob[j, pl.ds(0, 16)] = sb[j, pl.ds(0, 16)] * pv
                ob[j, pl.ds(16, 16)] = pv
                return 0

            lax.fori_loop(0, EB, edge, 0, unroll=4)
            pltpu.async_copy(ob, accum.at[idxd.at[b]], ssem, add=True)

        def run(cnt, start):
            pltpu.sync_copy(esrc_ref.at[pl.ds(start, cnt)],
                            idxs.at[pl.ds(0, cnt)])
            pltpu.sync_copy(edst_ref.at[pl.ds(start, cnt)],
                            idxd.at[pl.ds(0, cnt)])
            g_start(0, 0)

            def pair(g2, _):
                for i in range(2):
                    b = 2 * g2 + i

                    @pl.when(b < cnt)
                    def _():
                        g_wait(b, i)

                        @pl.when(b + 1 < cnt)
                        def _():
                            g_start(b + 1, 1 - i)

                        @pl.when(b >= 2)
                        def _():
                            s_wait(b - 2, i)

                        compute_block(b, i)
                return 0

            lax.fori_loop(0, (cnt + 1) // 2, pair, 0)
            s_wait(cnt - 2, (cnt - 2) % 2)
            s_wait(cnt - 1, (cnt - 1) % 2)

        @pl.when(c == 0)
        def _():
            run(B2C0, s * B2C0)

        @pl.when(c == 1)
        def _():
            run(B2C1, NS * B2C0 + s * B2C1)

        plsc.subcore_barrier()
        _dump_accum(accum, out_ref, c, base)

    return k(tab, as2, ad2, esrc, edst)


def kernel(x, edge_index, W1, a1_src, a1_dst, b1, W2, a2_src, a2_dst, b2):
    eye8 = jnp.eye(8, dtype=jnp.float32)
    AMs = jnp.pad((a1_src[:, :, None] * eye8[:, None, :]).reshape(64, 8),
                  ((0, 0), (0, 8)))
    AMd = jnp.pad((a1_dst[:, :, None] * eye8[:, None, :]).reshape(64, 8),
                  ((0, 0), (0, 8)))
    R = jnp.pad(jnp.repeat(eye8, 8, axis=1), ((0, 8), (0, 0)))
    ones16 = jnp.ones((1, 16), dtype=jnp.float32)
    Ms2 = a2_src.reshape(16, 1) @ ones16
    Md2 = a2_dst.reshape(16, 1) @ ones16

    xp = jnp.pad(x, ((0, NPAD - N), (0, 0)))
    tab1, adt1 = _tc1(xp, W1, AMs, AMd)

    ei = jnp.pad(edge_index, ((0, 0), (0, TBP * EB - E)), constant_values=N)
    esrc = ei[0].reshape(TBP, EB)
    edst = ei[1].reshape(TBP, EB)

    acc1 = _edge_pass1(tab1, adt1, esrc, edst)
    tab2, as2b, ad2b = _tc2(acc1, b1.reshape(1, 64), W2, Ms2, Md2, R)
    acc2 = _edge_pass2(tab2, as2b[:, 0], ad2b[:, 0], esrc, edst)
    out = _tc3(acc2, b2.reshape(1, 16))
    return out[:N]


# speedup vs baseline: 114.4583x; 1.0028x over previous
import functools

import jax
import jax.numpy as jnp
import numpy as np
from jax import lax
from jax.experimental import pallas as pl
from jax.experimental.pallas import tpu as pltpu
from jax.experimental.pallas import tpu_sc as plsc

N = 10000
E = 320000
F_IN = 128

NPAD = 10240
NC, NS = 2, 16
NW = NC * NS
EB = 128
B1C0, B1C1 = 103, 54
B2C0, B2C1 = 86, 71
NBT = B1C0 + B1C1
BMAX = max(B1C0, B1C1, B2C0, B2C1)
assert B2C0 + B2C1 == NBT
TBP = NS * NBT
ROWS_PER_SUB = NPAD // NS
RCHUNK = 128
NRC = ROWS_PER_SUB // RCHUNK
W1T = 80
W2T = 32

_NBLK = 1024
_GRID = NPAD // _NBLK

_SC_PARAMS = pltpu.CompilerParams(use_tc_tiling_on_sc=False,
                                  needs_layout_passes=False)


def _tc1_body(x_ref, w1_ref, ams_ref, amd_ref, tab_ref, adt_ref):
    h = jnp.dot(x_ref[...], w1_ref[...], preferred_element_type=jnp.float32)
    asb = jnp.dot(h, ams_ref[...], preferred_element_type=jnp.float32)
    adb = jnp.dot(h, amd_ref[...], preferred_element_type=jnp.float32)
    tab_ref[...] = jnp.concatenate([h, asb], axis=1)
    adt_ref[...] = adb


def _tc1(xp, W1, AMs, AMd):
    return pl.pallas_call(
        _tc1_body,
        grid=(_GRID,),
        in_specs=[
            pl.BlockSpec((_NBLK, F_IN), lambda i: (i, 0)),
            pl.BlockSpec((F_IN, 64), lambda i: (0, 0)),
            pl.BlockSpec((64, 16), lambda i: (0, 0)),
            pl.BlockSpec((64, 16), lambda i: (0, 0)),
        ],
        out_specs=[
            pl.BlockSpec((_NBLK, W1T), lambda i: (i, 0)),
            pl.BlockSpec((_NBLK, 16), lambda i: (i, 0)),
        ],
        out_shape=[
            jax.ShapeDtypeStruct((NPAD, W1T), jnp.float32),
            jax.ShapeDtypeStruct((NPAD, 16), jnp.float32),
        ],
    )(xp, W1, AMs, AMd)


def _tc2_body(acc_ref, b1_ref, w2_ref, ms_ref, md_ref, r_ref,
              tab_ref, as2_ref, ad2_ref):
    msg = acc_ref[0, :, :64] + acc_ref[1, :, :64]
    den16 = acc_ref[0, :, 64:80] + acc_ref[1, :, 64:80]
    den = jnp.dot(den16, r_ref[...], preferred_element_type=jnp.float32)
    hl = msg / (den + 1e-16) + b1_ref[...]
    hl = jnp.where(hl > 0, hl, jnp.exp(hl) - 1.0)
    h2 = jnp.dot(hl, w2_ref[...], preferred_element_type=jnp.float32)
    asb = jnp.dot(h2, ms_ref[...], preferred_element_type=jnp.float32)
    adb = jnp.dot(h2, md_ref[...], preferred_element_type=jnp.float32)
    tab_ref[...] = h2
    as2_ref[...] = asb
    ad2_ref[...] = adb


def _tc2(acc1, b1, W2, Ms2, Md2, R):
    return pl.pallas_call(
        _tc2_body,
        grid=(_GRID,),
        in_specs=[
            pl.BlockSpec((2, _NBLK, W1T), lambda i: (0, i, 0)),
            pl.BlockSpec((1, 64), lambda i: (0, 0)),
            pl.BlockSpec((64, 16), lambda i: (0, 0)),
            pl.BlockSpec((16, 16), lambda i: (0, 0)),
            pl.BlockSpec((16, 16), lambda i: (0, 0)),
            pl.BlockSpec((16, 64), lambda i: (0, 0)),
        ],
        out_specs=[
            pl.BlockSpec((_NBLK, 16), lambda i: (i, 0)),
            pl.BlockSpec((_NBLK, 16), lambda i: (i, 0)),
            pl.BlockSpec((_NBLK, 16), lambda i: (i, 0)),
        ],
        out_shape=[
            jax.ShapeDtypeStruct((NPAD, 16), jnp.float32),
            jax.ShapeDtypeStruct((NPAD, 16), jnp.float32),
            jax.ShapeDtypeStruct((NPAD, 16), jnp.float32),
        ],
    )(acc1, b1, W2, Ms2, Md2, R)


def _tc3_body(acc_ref, b2_ref, out_ref):
    msg = acc_ref[0, :, :16] + acc_ref[1, :, :16]
    den = acc_ref[0, :, 16:32] + acc_ref[1, :, 16:32]
    o = msg / (den + 1e-16) + b2_ref[...]
    m = jnp.max(o, axis=1, keepdims=True)
    z = o - m
    out_ref[...] = z - jnp.log(jnp.sum(jnp.exp(z), axis=1, keepdims=True))


def _tc3(acc2, b2):
    return pl.pallas_call(
        _tc3_body,
        grid=(_GRID,),
        in_specs=[
            pl.BlockSpec((2, _NBLK, W2T), lambda i: (0, i, 0)),
            pl.BlockSpec((1, 16), lambda i: (0, 0)),
        ],
        out_specs=pl.BlockSpec((_NBLK, 16), lambda i: (i, 0)),
        out_shape=jax.ShapeDtypeStruct((NPAD, 16), jnp.float32),
    )(acc2, b2)


_GDN = lax.GatherDimensionNumbers(
    offset_dims=(), collapsed_slice_dims=(0,), start_index_map=(0,))


def _pair_idx():
    io = lax.iota(jnp.int32, 16)
    hi = jnp.where(io >= 8, 1, 0)
    return [(2 * k + hi).reshape(16, 1) for k in range(4)]


def _bcast_pair(p, idx):
    return lax.gather(p, idx, _GDN, (1,),
                      mode=lax.GatherScatterMode.PROMISE_IN_BOUNDS)


def _leaky_exp(e):
    return jnp.exp(jnp.maximum(e, e * 0.2))


def _zero_accum(orows, accum, s, w):
    zv = jnp.zeros((16,), jnp.float32)

    def zrow(i, _):
        for kk in range(w // 16):
            orows[i, pl.ds(kk * 16, 16)] = zv
        return 0

    lax.fori_loop(0, EB, zrow, 0)
    base = s * ROWS_PER_SUB
    for jj in range(NRC):
        pltpu.sync_copy(orows, accum.at[pl.ds(base + jj * RCHUNK, RCHUNK)])
    return base


def _dump_accum(accum, out_ref, c, base):
    for jj in range(NRC):
        r0 = base + jj * RCHUNK
        pltpu.sync_copy(accum.at[pl.ds(r0, RCHUNK)],
                        out_ref.at[c, pl.ds(r0, RCHUNK)])


def _edge_pass1(tab, adt, esrc, edst):
    mesh = plsc.VectorSubcoreMesh(core_axis_name="c", subcore_axis_name="s")

    @functools.partial(
        pl.kernel,
        out_type=jax.ShapeDtypeStruct((NC, NPAD, W1T), jnp.float32),
        mesh=mesh,
        compiler_params=_SC_PARAMS,
        scratch_types=[
            pltpu.VMEM((BMAX, EB), jnp.int32),
            pltpu.VMEM((BMAX, EB), jnp.int32),
            pltpu.VMEM((EB, W1T), jnp.float32),
            pltpu.VMEM((EB, W1T), jnp.float32),
            pltpu.VMEM((EB, 16), jnp.float32),
            pltpu.VMEM((EB, 16), jnp.float32),
            pltpu.VMEM((EB, W1T), jnp.float32),
            pltpu.VMEM((EB, W1T), jnp.float32),
            pltpu.SemaphoreType.DMA,
            pltpu.SemaphoreType.DMA,
            pltpu.SemaphoreType.DMA,
            pltpu.SemaphoreType.DMA,
            pltpu.VMEM_SHARED((NPAD, W1T), jnp.float32),
        ],
    )
    def k(tab_ref, adt_ref, esrc_ref, edst_ref, out_ref,
          idxs, idxd, srows0, srows1, drows0, drows1, orows0, orows1,
          gsem0, gsem1, ssem0, ssem1, accum):
        c = lax.axis_index("c")
        s = lax.axis_index("s")
        base = _zero_accum(orows0, accum, s, W1T)
        plsc.subcore_barrier()

        bufs = ((srows0, drows0, orows0, gsem0, ssem0),
                (srows1, drows1, orows1, gsem1, ssem1))
        pidx = _pair_idx()

        def g_start(b, i):
            sb, db, _, sem, _ = bufs[i]
            pltpu.async_copy(tab_ref.at[idxs.at[b]], sb, sem)
            pltpu.async_copy(adt_ref.at[idxd.at[b]], db, sem)

        def g_wait(b, i):
            sb, db, _, sem, _ = bufs[i]
            pltpu.make_async_copy(tab_ref.at[idxs.at[b]], sb, sem).wait()
            pltpu.make_async_copy(adt_ref.at[idxd.at[b]], db, sem).wait()

        def s_wait(b, i):
            _, _, ob, _, sem = bufs[i]
            pltpu.make_async_copy(ob, accum.at[idxd.at[b]], sem).wait()

        def compute_block(b, i):
            sb, db, ob, _, ssem = bufs[i]

            def compute_p(j):
                sa = sb[j, pl.ds(64, 16)]
                da = db[j, pl.ds(0, 16)]
                return _leaky_exp(sa + da)

            def do_msg(j, p):
                for kk in range(4):
                    hk = sb[j, pl.ds(kk * 16, 16)]
                    ob[j, pl.ds(kk * 16, 16)] = hk * _bcast_pair(p, pidx[kk])
                ob[j, pl.ds(64, 16)] = p

            def edge(j, carry):
                p1, p2 = carry
                p_new = compute_p(j)
                do_msg(j - 2, p2)
                return (p_new, p1)

            pl_, pl2 = lax.fori_loop(2, EB, edge,
                                     (compute_p(1), compute_p(0)), unroll=8)
            do_msg(EB - 2, pl2)
            do_msg(EB - 1, pl_)
            pltpu.async_copy(ob, accum.at[idxd.at[b]], ssem, add=True)

        def run(cnt, start):
            pltpu.sync_copy(esrc_ref.at[pl.ds(start, cnt)],
                            idxs.at[pl.ds(0, cnt)])
            pltpu.sync_copy(edst_ref.at[pl.ds(start, cnt)],
                            idxd.at[pl.ds(0, cnt)])
            g_start(0, 0)

            def pair(g2, _):
                for i in range(2):
                    b = 2 * g2 + i

                    @pl.when(b < cnt)
                    def _():
                        g_wait(b, i)

                        @pl.when(b + 1 < cnt)
                        def _():
                            g_start(b + 1, 1 - i)

                        @pl.when(b >= 2)
                        def _():
                            s_wait(b - 2, i)

                        compute_block(b, i)
                return 0

            lax.fori_loop(0, (cnt + 1) // 2, pair, 0)
            s_wait(cnt - 2, (cnt - 2) % 2)
            s_wait(cnt - 1, (cnt - 1) % 2)

        @pl.when(c == 0)
        def _():
            run(B1C0, s * B1C0)

        @pl.when(c == 1)
        def _():
            run(B1C1, NS * B1C0 + s * B1C1)

        plsc.subcore_barrier()
        _dump_accum(accum, out_ref, c, base)

    return k(tab, adt, esrc, edst)


def _edge_pass2(tab, as2, ad2, esrc, edst):
    mesh = plsc.VectorSubcoreMesh(core_axis_name="c", subcore_axis_name="s")

    @functools.partial(
        pl.kernel,
        out_type=jax.ShapeDtypeStruct((NC, NPAD, W2T), jnp.float32),
        mesh=mesh,
        compiler_params=_SC_PARAMS,
        scratch_types=[
            pltpu.VMEM((BMAX, EB), jnp.int32),
            pltpu.VMEM((BMAX, EB), jnp.int32),
            pltpu.VMEM((EB, 16), jnp.float32),
            pltpu.VMEM((EB, 16), jnp.float32),
            pltpu.VMEM((EB, W2T), jnp.float32),
            pltpu.VMEM((EB, W2T), jnp.float32),
            pltpu.VMEM((NPAD,), jnp.float32),
            pltpu.VMEM((NPAD,), jnp.float32),
            pltpu.VMEM((EB,), jnp.float32),
            pltpu.SemaphoreType.DMA,
            pltpu.SemaphoreType.DMA,
            pltpu.SemaphoreType.DMA,
            pltpu.SemaphoreType.DMA,
            pltpu.VMEM_SHARED((NPAD, W2T), jnp.float32),
        ],
    )
    def k(tab_ref, as2_ref, ad2_ref, esrc_ref, edst_ref, out_ref,
          idxs, idxd, srows0, srows1, orows0, orows1, asv, adv, pbuf,
          gsem0, gsem1, ssem0, ssem1, accum):
        c = lax.axis_index("c")
        s = lax.axis_index("s")
        pltpu.sync_copy(as2_ref, asv)
        pltpu.sync_copy(ad2_ref, adv)
        base = _zero_accum(orows0, accum, s, W2T)
        plsc.subcore_barrier()

        bufs = ((srows0, orows0, gsem0, ssem0), (srows1, orows1, gsem1, ssem1))

        def g_start(b, i):
            sb, _, sem, _ = bufs[i]
            pltpu.async_copy(tab_ref.at[idxs.at[b]], sb, sem)

        def g_wait(b, i):
            sb, _, sem, _ = bufs[i]
            pltpu.make_async_copy(tab_ref.at[idxs.at[b]], sb, sem).wait()

        def s_wait(b, i):
            _, ob, _, sem = bufs[i]
            pltpu.make_async_copy(ob, accum.at[idxd.at[b]], sem).wait()

        def compute_block(b, i):
            sb, ob, _, ssem = bufs[i]

            def grp(g, _):
                srcv = idxs[b, pl.ds(g * 16, 16)]
                dstv = idxd[b, pl.ds(g * 16, 16)]
                e = (plsc.load_gather(asv, [srcv]) +
                     plsc.load_gather(adv, [dstv]))
                pbuf[pl.ds(g * 16, 16)] = _leaky_exp(e)
                return 0

            lax.fori_loop(0, EB // 16, grp, 0, unroll=2)

            def edge(j, _):
                pv = plsc.load_gather(pbuf, [jnp.broadcast_to(j, (16,))])
                ob[j, pl.ds(0, 16)] = sb[j, pl.ds(0, 16)] * pv
                ob[j, pl.ds(16, 16)] = pv
                return 0

            lax.fori_loop(0, EB, edge, 0, unroll=4)
            pltpu.async_copy(ob, accum.at[idxd.at[b]], ssem, add=True)

        def run(cnt, start):
            pltpu.sync_copy(esrc_ref.at[pl.ds(start, cnt)],
                            idxs.at[pl.ds(0, cnt)])
            pltpu.sync_copy(edst_ref.at[pl.ds(start, cnt)],
                            idxd.at[pl.ds(0, cnt)])
            g_start(0, 0)

            def pair(g2, _):
                for i in range(2):
                    b = 2 * g2 + i

                    @pl.when(b < cnt)
                    def _():
                        g_wait(b, i)

                        @pl.when(b + 1 < cnt)
                        def _():
                            g_start(b + 1, 1 - i)

                        @pl.when(b >= 2)
                        def _():
                            s_wait(b - 2, i)

                        compute_block(b, i)
                return 0

            lax.fori_loop(0, (cnt + 1) // 2, pair, 0)
            s_wait(cnt - 2, (cnt - 2) % 2)
            s_wait(cnt - 1, (cnt - 1) % 2)

        @pl.when(c == 0)
        def _():
            run(B2C0, s * B2C0)

        @pl.when(c == 1)
        def _():
            run(B2C1, NS * B2C0 + s * B2C1)

        plsc.subcore_barrier()
        _dump_accum(accum, out_ref, c, base)

    return k(tab, as2, ad2, esrc, edst)


def kernel(x, edge_index, W1, a1_src, a1_dst, b1, W2, a2_src, a2_dst, b2):
    eye8 = jnp.eye(8, dtype=jnp.float32)
    AMs = jnp.pad((a1_src[:, :, None] * eye8[:, None, :]).reshape(64, 8),
                  ((0, 0), (0, 8)))
    AMd = jnp.pad((a1_dst[:, :, None] * eye8[:, None, :]).reshape(64, 8),
                  ((0, 0), (0, 8)))
    R = jnp.pad(jnp.repeat(eye8, 8, axis=1), ((0, 8), (0, 0)))
    ones16 = jnp.ones((1, 16), dtype=jnp.float32)
    Ms2 = a2_src.reshape(16, 1) @ ones16
    Md2 = a2_dst.reshape(16, 1) @ ones16

    xp = jnp.pad(x, ((0, NPAD - N), (0, 0)))
    tab1, adt1 = _tc1(xp, W1, AMs, AMd)

    ei = jnp.pad(edge_index, ((0, 0), (0, TBP * EB - E)), constant_values=N)
    esrc = ei[0].reshape(TBP, EB)
    edst = ei[1].reshape(TBP, EB)

    acc1 = _edge_pass1(tab1, adt1, esrc, edst)
    tab2, as2b, ad2b = _tc2(acc1, b1.reshape(1, 64), W2, Ms2, Md2, R)
    acc2 = _edge_pass2(tab2, as2b[:, 0], ad2b[:, 0], esrc, edst)
    out = _tc3(acc2, b2.reshape(1, 16))
    return out[:N]


# speedup vs baseline: 122.2466x; 1.0680x over previous
import functools

import jax
import jax.numpy as jnp
import numpy as np
from jax import lax
from jax.experimental import pallas as pl
from jax.experimental.pallas import tpu as pltpu
from jax.experimental.pallas import tpu_sc as plsc

N = 10000
E = 320000
F_IN = 128

NPAD = 10240
NC, NS = 2, 16
NW = NC * NS
EB = 128
B1C0, B1C1 = 94, 63
B2C0, B2C1 = 84, 73
NBT = B1C0 + B1C1
BMAX = max(B1C0, B1C1, B2C0, B2C1)
assert B2C0 + B2C1 == NBT
TBP = NS * NBT
ROWS_PER_SUB = NPAD // NS
RCHUNK = 128
NRC = ROWS_PER_SUB // RCHUNK
W1T = 80
STW = 72
W2T = 32

_NBLK = 2048
_GRID = NPAD // _NBLK

_SC_PARAMS = pltpu.CompilerParams(use_tc_tiling_on_sc=False,
                                  needs_layout_passes=False)


def _tc1_body(x_ref, w1_ref, ams_ref, amd_ref, tab_ref, adt_ref):
    h = jnp.dot(x_ref[...], w1_ref[...], preferred_element_type=jnp.float32)
    asb = jnp.dot(h, ams_ref[...], preferred_element_type=jnp.float32)
    adb = jnp.dot(h, amd_ref[...], preferred_element_type=jnp.float32)
    tab_ref[...] = jnp.concatenate([h, asb], axis=1)
    adt_ref[...] = adb


def _tc1(xp, W1, AMs, AMd):
    return pl.pallas_call(
        _tc1_body,
        grid=(_GRID,),
        in_specs=[
            pl.BlockSpec((_NBLK, F_IN), lambda i: (i, 0)),
            pl.BlockSpec((F_IN, 64), lambda i: (0, 0)),
            pl.BlockSpec((64, 8), lambda i: (0, 0)),
            pl.BlockSpec((64, 16), lambda i: (0, 0)),
        ],
        out_specs=[
            pl.BlockSpec((_NBLK, STW), lambda i: (i, 0)),
            pl.BlockSpec((_NBLK, 16), lambda i: (i, 0)),
        ],
        out_shape=[
            jax.ShapeDtypeStruct((NPAD, STW), jnp.float32),
            jax.ShapeDtypeStruct((NPAD, 16), jnp.float32),
        ],
    )(xp, W1, AMs, AMd)


def _tc2_body(acc_ref, b1_ref, w2_ref, ms_ref, md_ref, r_ref,
              tab_ref, as2_ref, ad2_ref):
    msg = acc_ref[0, :, :64] + acc_ref[1, :, :64]
    den16 = acc_ref[0, :, 64:80] + acc_ref[1, :, 64:80]
    den = jnp.dot(den16, r_ref[...], preferred_element_type=jnp.float32)
    hl = msg / (den + 1e-16) + b1_ref[...]
    hl = jnp.where(hl > 0, hl, jnp.exp(hl) - 1.0)
    h2 = jnp.dot(hl, w2_ref[...], preferred_element_type=jnp.float32)
    asb = jnp.dot(h2, ms_ref[...], preferred_element_type=jnp.float32)
    adb = jnp.dot(h2, md_ref[...], preferred_element_type=jnp.float32)
    tab_ref[...] = h2
    as2_ref[...] = asb
    ad2_ref[...] = adb


def _tc2(acc1, b1, W2, Ms2, Md2, R):
    return pl.pallas_call(
        _tc2_body,
        grid=(_GRID,),
        in_specs=[
            pl.BlockSpec((2, _NBLK, W1T), lambda i: (0, i, 0)),
            pl.BlockSpec((1, 64), lambda i: (0, 0)),
            pl.BlockSpec((64, 16), lambda i: (0, 0)),
            pl.BlockSpec((16, 16), lambda i: (0, 0)),
            pl.BlockSpec((16, 16), lambda i: (0, 0)),
            pl.BlockSpec((16, 64), lambda i: (0, 0)),
        ],
        out_specs=[
            pl.BlockSpec((_NBLK, 16), lambda i: (i, 0)),
            pl.BlockSpec((_NBLK, 16), lambda i: (i, 0)),
            pl.BlockSpec((_NBLK, 16), lambda i: (i, 0)),
        ],
        out_shape=[
            jax.ShapeDtypeStruct((NPAD, 16), jnp.float32),
            jax.ShapeDtypeStruct((NPAD, 16), jnp.float32),
            jax.ShapeDtypeStruct((NPAD, 16), jnp.float32),
        ],
    )(acc1, b1, W2, Ms2, Md2, R)


def _tc3_body(acc_ref, b2_ref, out_ref):
    msg = acc_ref[0, :, :16] + acc_ref[1, :, :16]
    den = acc_ref[0, :, 16:32] + acc_ref[1, :, 16:32]
    o = msg / (den + 1e-16) + b2_ref[...]
    m = jnp.max(o, axis=1, keepdims=True)
    z = o - m
    out_ref[...] = z - jnp.log(jnp.sum(jnp.exp(z), axis=1, keepdims=True))


def _tc3(acc2, b2):
    return pl.pallas_call(
        _tc3_body,
        grid=(_GRID,),
        in_specs=[
            pl.BlockSpec((2, _NBLK, W2T), lambda i: (0, i, 0)),
            pl.BlockSpec((1, 16), lambda i: (0, 0)),
        ],
        out_specs=pl.BlockSpec((_NBLK, 16), lambda i: (i, 0)),
        out_shape=jax.ShapeDtypeStruct((NPAD, 16), jnp.float32),
    )(acc2, b2)


_GDN = lax.GatherDimensionNumbers(
    offset_dims=(), collapsed_slice_dims=(0,), start_index_map=(0,))


def _pair_idx():
    io = lax.iota(jnp.int32, 16)
    hi = jnp.where(io >= 8, 1, 0)
    return [(8 + 2 * k + hi).reshape(16, 1) for k in range(4)]


def _bcast_pair(p, idx):
    return lax.gather(p, idx, _GDN, (1,),
                      mode=lax.GatherScatterMode.PROMISE_IN_BOUNDS)


def _leaky_exp(e):
    return jnp.exp(jnp.maximum(e, e * 0.2))


def _zero_accum(orows, accum, s, w):
    zv = jnp.zeros((16,), jnp.float32)

    def zrow(i, _):
        for kk in range(w // 16):
            orows[i, pl.ds(kk * 16, 16)] = zv
        return 0

    lax.fori_loop(0, EB, zrow, 0)
    base = s * ROWS_PER_SUB
    for jj in range(NRC):
        pltpu.sync_copy(orows, accum.at[pl.ds(base + jj * RCHUNK, RCHUNK)])
    return base


def _dump_accum(accum, out_ref, c, base):
    for jj in range(NRC):
        r0 = base + jj * RCHUNK
        pltpu.sync_copy(accum.at[pl.ds(r0, RCHUNK)],
                        out_ref.at[c, pl.ds(r0, RCHUNK)])


def _edge_pass1(tab, adt, esrc, edst):
    mesh = plsc.VectorSubcoreMesh(core_axis_name="c", subcore_axis_name="s")

    @functools.partial(
        pl.kernel,
        out_type=jax.ShapeDtypeStruct((NC, NPAD, W1T), jnp.float32),
        mesh=mesh,
        compiler_params=_SC_PARAMS,
        scratch_types=[
            pltpu.VMEM((BMAX, EB), jnp.int32),
            pltpu.VMEM((BMAX, EB), jnp.int32),
            pltpu.VMEM((EB + 1, STW), jnp.float32),
            pltpu.VMEM((EB + 1, STW), jnp.float32),
            pltpu.VMEM((EB, 16), jnp.float32),
            pltpu.VMEM((EB, 16), jnp.float32),
            pltpu.VMEM((EB, W1T), jnp.float32),
            pltpu.VMEM((EB, W1T), jnp.float32),
            pltpu.SemaphoreType.DMA,
            pltpu.SemaphoreType.DMA,
            pltpu.SemaphoreType.DMA,
            pltpu.SemaphoreType.DMA,
            pltpu.VMEM_SHARED((NPAD, W1T), jnp.float32),
        ],
    )
    def k(tab_ref, adt_ref, esrc_ref, edst_ref, out_ref,
          idxs, idxd, srows0, srows1, drows0, drows1, orows0, orows1,
          gsem0, gsem1, ssem0, ssem1, accum):
        c = lax.axis_index("c")
        s = lax.axis_index("s")
        base = _zero_accum(orows0, accum, s, W1T)
        plsc.subcore_barrier()

        bufs = ((srows0, drows0, orows0, gsem0, ssem0),
                (srows1, drows1, orows1, gsem1, ssem1))
        pidx = _pair_idx()

        def g_start(b, i):
            sb, db, _, sem, _ = bufs[i]
            pltpu.async_copy(tab_ref.at[idxs.at[b]], sb.at[pl.ds(0, EB)], sem)
            pltpu.async_copy(adt_ref.at[idxd.at[b]], db, sem)

        def g_wait(b, i):
            sb, db, _, sem, _ = bufs[i]
            pltpu.make_async_copy(tab_ref.at[idxs.at[b]],
                                  sb.at[pl.ds(0, EB)], sem).wait()
            pltpu.make_async_copy(adt_ref.at[idxd.at[b]], db, sem).wait()

        def s_wait(b, i):
            _, _, ob, _, sem = bufs[i]
            pltpu.make_async_copy(ob, accum.at[idxd.at[b]], sem).wait()

        def compute_block(b, i):
            sb, db, ob, _, ssem = bufs[i]

            def compute_p(j):
                sa = sb[j, pl.ds(56, 16)]
                da = db[j, pl.ds(0, 16)]
                return _leaky_exp(sa + da)

            def do_msg(j, p):
                for kk in range(4):
                    hk = sb[j, pl.ds(kk * 16, 16)]
                    ob[j, pl.ds(kk * 16, 16)] = hk * _bcast_pair(p, pidx[kk])
                ob[j, pl.ds(64, 16)] = p

            def edge(j, carry):
                p1, p2 = carry
                p_new = compute_p(j)
                do_msg(j - 2, p2)
                return (p_new, p1)

            pl_, pl2 = lax.fori_loop(2, EB, edge,
                                     (compute_p(1), compute_p(0)), unroll=8)
            do_msg(EB - 2, pl2)
            do_msg(EB - 1, pl_)
            pltpu.async_copy(ob, accum.at[idxd.at[b]], ssem, add=True)

        def run(cnt, start):
            pltpu.sync_copy(esrc_ref.at[pl.ds(start, cnt)],
                            idxs.at[pl.ds(0, cnt)])
            pltpu.sync_copy(edst_ref.at[pl.ds(start, cnt)],
                            idxd.at[pl.ds(0, cnt)])
            g_start(0, 0)

            def pair(g2, _):
                for i in range(2):
                    b = 2 * g2 + i

                    @pl.when(b < cnt)
                    def _():
                        g_wait(b, i)

                        @pl.when(b + 1 < cnt)
                        def _():
                            g_start(b + 1, 1 - i)

                        @pl.when(b >= 2)
                        def _():
                            s_wait(b - 2, i)

                        compute_block(b, i)
                return 0

            lax.fori_loop(0, (cnt + 1) // 2, pair, 0)
            s_wait(cnt - 2, (cnt - 2) % 2)
            s_wait(cnt - 1, (cnt - 1) % 2)

        @pl.when(c == 0)
        def _():
            run(B1C0, s * B1C0)

        @pl.when(c == 1)
        def _():
            run(B1C1, NS * B1C0 + s * B1C1)

        plsc.subcore_barrier()
        _dump_accum(accum, out_ref, c, base)

    return k(tab, adt, esrc, edst)


def _edge_pass2(tab, as2, ad2, esrc, edst):
    mesh = plsc.VectorSubcoreMesh(core_axis_name="c", subcore_axis_name="s")

    @functools.partial(
        pl.kernel,
        out_type=jax.ShapeDtypeStruct((NC, NPAD, W2T), jnp.float32),
        mesh=mesh,
        compiler_params=_SC_PARAMS,
        scratch_types=[
            pltpu.VMEM((BMAX, EB), jnp.int32),
            pltpu.VMEM((BMAX, EB), jnp.int32),
            pltpu.VMEM((EB, 16), jnp.float32),
            pltpu.VMEM((EB, 16), jnp.float32),
            pltpu.VMEM((EB, W2T), jnp.float32),
            pltpu.VMEM((EB, W2T), jnp.float32),
            pltpu.VMEM((NPAD,), jnp.float32),
            pltpu.VMEM((NPAD,), jnp.float32),
            pltpu.VMEM((EB,), jnp.float32),
            pltpu.SemaphoreType.DMA,
            pltpu.SemaphoreType.DMA,
            pltpu.SemaphoreType.DMA,
            pltpu.SemaphoreType.DMA,
            pltpu.VMEM_SHARED((NPAD, W2T), jnp.float32),
        ],
    )
    def k(tab_ref, as2_ref, ad2_ref, esrc_ref, edst_ref, out_ref,
          idxs, idxd, srows0, srows1, orows0, orows1, asv, adv, pbuf,
          gsem0, gsem1, ssem0, ssem1, accum):
        c = lax.axis_index("c")
        s = lax.axis_index("s")
        pltpu.sync_copy(as2_ref, asv)
        pltpu.sync_copy(ad2_ref, adv)
        base = _zero_accum(orows0, accum, s, W2T)
        plsc.subcore_barrier()

        bufs = ((srows0, orows0, gsem0, ssem0), (srows1, orows1, gsem1, ssem1))

        def g_start(b, i):
            sb, _, sem, _ = bufs[i]
            pltpu.async_copy(tab_ref.at[idxs.at[b]], sb, sem)

        def g_wait(b, i):
            sb, _, sem, _ = bufs[i]
            pltpu.make_async_copy(tab_ref.at[idxs.at[b]], sb, sem).wait()

        def s_wait(b, i):
            _, ob, _, sem = bufs[i]
            pltpu.make_async_copy(ob, accum.at[idxd.at[b]], sem).wait()

        def compute_block(b, i):
            sb, ob, _, ssem = bufs[i]

            def grp(g, _):
                srcv = idxs[b, pl.ds(g * 16, 16)]
                dstv = idxd[b, pl.ds(g * 16, 16)]
                e = (plsc.load_gather(asv, [srcv]) +
                     plsc.load_gather(adv, [dstv]))
                pbuf[pl.ds(g * 16, 16)] = _leaky_exp(e)
                return 0

            lax.fori_loop(0, EB // 16, grp, 0, unroll=2)

            def edge(j, _):
                pv = plsc.load_gather(pbuf, [jnp.broadcast_to(j, (16,))])
                ob[j, pl.ds(0, 16)] = sb[j, pl.ds(0, 16)] * pv
                ob[j, pl.ds(16, 16)] = pv
                return 0

            lax.fori_loop(0, EB, edge, 0, unroll=4)
            pltpu.async_copy(ob, accum.at[idxd.at[b]], ssem, add=True)

        def run(cnt, start):
            pltpu.sync_copy(esrc_ref.at[pl.ds(start, cnt)],
                            idxs.at[pl.ds(0, cnt)])
            pltpu.sync_copy(edst_ref.at[pl.ds(start, cnt)],
                            idxd.at[pl.ds(0, cnt)])
            g_start(0, 0)

            def pair(g2, _):
                for i in range(2):
                    b = 2 * g2 + i

                    @pl.when(b < cnt)
                    def _():
                        g_wait(b, i)

                        @pl.when(b + 1 < cnt)
                        def _():
                            g_start(b + 1, 1 - i)

                        @pl.when(b >= 2)
                        def _():
                            s_wait(b - 2, i)

                        compute_block(b, i)
                return 0

            lax.fori_loop(0, (cnt + 1) // 2, pair, 0)
            s_wait(cnt - 2, (cnt - 2) % 2)
            s_wait(cnt - 1, (cnt - 1) % 2)

        @pl.when(c == 0)
        def _():
            run(B2C0, s * B2C0)

        @pl.when(c == 1)
        def _():
            run(B2C1, NS * B2C0 + s * B2C1)

        plsc.subcore_barrier()
        _dump_accum(accum, out_ref, c, base)

    return k(tab, as2, ad2, esrc, edst)


def kernel(x, edge_index, W1, a1_src, a1_dst, b1, W2, a2_src, a2_dst, b2):
    eye8 = jnp.eye(8, dtype=jnp.float32)
    AMs = (a1_src[:, :, None] * eye8[:, None, :]).reshape(64, 8)
    AMd = jnp.pad((a1_dst[:, :, None] * eye8[:, None, :]).reshape(64, 8),
                  ((0, 0), (8, 0)))
    R = jnp.pad(jnp.repeat(eye8, 8, axis=1), ((8, 0), (0, 0)))
    ones16 = jnp.ones((1, 16), dtype=jnp.float32)
    Ms2 = a2_src.reshape(16, 1) @ ones16
    Md2 = a2_dst.reshape(16, 1) @ ones16

    xp = jnp.pad(x, ((0, NPAD - N), (0, 0)))
    tab1, adt1 = _tc1(xp, W1, AMs, AMd)

    ei = jnp.pad(edge_index, ((0, 0), (0, TBP * EB - E)), constant_values=N)
    esrc = ei[0].reshape(TBP, EB)
    edst = ei[1].reshape(TBP, EB)

    acc1 = _edge_pass1(tab1, adt1, esrc, edst)
    tab2, as2b, ad2b = _tc2(acc1, b1.reshape(1, 64), W2, Ms2, Md2, R)
    acc2 = _edge_pass2(tab2, as2b[:, 0], ad2b[:, 0], esrc, edst)
    out = _tc3(acc2, b2.reshape(1, 16))
    return out[:N]
